# baseline (speedup 1.0000x reference)
import functools

import jax
import jax.numpy as jnp
import numpy as np
from jax import lax
from jax.experimental import pallas as pl
from jax.experimental.pallas import tpu as pltpu
from jax.experimental.pallas import tpu_sc as plsc

_C = 16
_NCELL = _C * _C * _C
_NP = 16384
_NQ = 32768
_K = 10
_R2 = np.float32(0.0625)
_INF = np.float32(np.inf)
_CELL2 = np.float32(1.0 / (_C * _C * _C * _C))
_NW = 32
_QPW = _NQ // _NW
_HALF = _QPW // 2

_cols = []
for _dx in range(-5, 6):
    for _dy in range(-5, 6):
        _m1 = max(abs(_dx) - 1, 0)
        _m2 = max(abs(_dy) - 1, 0)
        _mm = _m1 * _m1 + _m2 * _m2
        if _mm <= 22:
            _cols.append((_mm, _dx, _dy))
_cols.sort()
_NCOL = len(_cols)
_NCOLP = ((_NCOL + 7) // 8) * 8
_CDX = np.array([c[1] for c in _cols] + [0] * (_NCOLP - _NCOL), np.int32)
_CDY = np.array([c[2] for c in _cols] + [0] * (_NCOLP - _NCOL), np.int32)
_CM2 = np.array([c[0] for c in _cols] + [0] * (_NCOLP - _NCOL), np.int32)
_CLB2 = np.array(
    [c[0] * float(_CELL2) for c in _cols] + [np.inf] * (_NCOLP - _NCOL),
    np.float32)
_ZLUT = np.array([int(np.floor(np.sqrt(r))) for r in range(24)], np.int32)
_CCNT = np.array([sum(1 for c in _cols if c[0] <= t) for t in range(24)],
                 np.int32)

_IOTA = None


def _sload(ref, i):
    return ref[pl.ds(i, 16)][0]


def _rne_bf16(v):
    b = lax.bitcast_convert_type(v, jnp.uint32)
    r = (b + jnp.uint32(0x7FFF) + ((b >> jnp.uint32(16)) & jnp.uint32(1)))
    r = r & jnp.uint32(0xFFFF0000)
    return lax.bitcast_convert_type(r, jnp.float32)


def _sc_body(spx_h, spy_h, spz_h, pidx_h, starts_h, qx_h, qy_h, qz_h,
             cdx_h, cdy_h, cm2_h, clb2_h, lut_h, ccnt_h,
             omap_h, ox_h, oy_h, oz_h,
             px_v, py_v, pz_v, pn_v, pidx_v, starts_v,
             qx_v, qy_v, qz_v,
             cdx_v, cdy_v, cm2_v, clb2_v, lut_v, ccnt_v,
             oi_v, ox_v, oy_v, oz_v):
    wid = lax.axis_index("s") * 2 + lax.axis_index("c")
    qbase = wid * _QPW

    pltpu.sync_copy(spx_h, px_v)
    pltpu.sync_copy(spy_h, py_v)
    pltpu.sync_copy(spz_h, pz_v)
    pltpu.sync_copy(pidx_h, pidx_v)
    pltpu.sync_copy(starts_h, starts_v.at[pl.ds(0, _NCELL + 8)])
    pltpu.sync_copy(qx_h.at[pl.ds(qbase, _QPW)], qx_v.at[pl.ds(0, _QPW)])
    pltpu.sync_copy(qy_h.at[pl.ds(qbase, _QPW)], qy_v.at[pl.ds(0, _QPW)])
    pltpu.sync_copy(qz_h.at[pl.ds(qbase, _QPW)], qz_v.at[pl.ds(0, _QPW)])
    pltpu.sync_copy(cdx_h, cdx_v.at[pl.ds(0, _NCOLP)])
    pltpu.sync_copy(cdy_h, cdy_v.at[pl.ds(0, _NCOLP)])
    pltpu.sync_copy(cm2_h, cm2_v.at[pl.ds(0, _NCOLP)])
    pltpu.sync_copy(clb2_h, clb2_v.at[pl.ds(0, _NCOLP)])
    pltpu.sync_copy(lut_h, lut_v.at[pl.ds(0, 24)])
    pltpu.sync_copy(ccnt_h, ccnt_v.at[pl.ds(0, 24)])

    iota = lax.iota(jnp.int32, 16)
    rank_mask = iota < _K

    def _pn_step(i, smax):
        sl = pl.ds(i * 16, 16)
        px = px_v[sl]
        py = py_v[sl]
        pz = pz_v[sl]
        pn_v[sl] = (px * px + pz * pz) + py * py
        sp = (jnp.abs(px - _rne_bf16(px)) + jnp.abs(py - _rne_bf16(py))
              + jnp.abs(pz - _rne_bf16(pz)))
        return jnp.maximum(smax, sp)

    smax_vec = lax.fori_loop(0, _NP // 16, _pn_step,
                             jnp.zeros((16,), jnp.float32))
    for _sh in (8, 4, 2, 1):
        smax_vec = jnp.maximum(smax_vec, jnp.take(smax_vec, iota ^ _sh))
    e_base = 2.0 * smax_vec[0] + np.float32(1e-6)

    def _merge(keys, vals, nk, nv):
        sk, sv = plsc.sort_key_val(nk, nv)
        rk = lax.rev(sk, (0,))
        rv = lax.rev(sv, (0,))
        take = keys <= rk
        mk = jnp.where(take, keys, rk)
        mv = jnp.where(take, vals, rv)
        out = plsc.sort_key_val(mk, mv)
        return out[0], out[1]

    def _w10(keys):
        return keys[9]

    def _do_query(qi, ql):
        qx = _sload(qx_v, qi)
        qy = _sload(qy_v, qi)
        qz = _sload(qz_v, qi)
        qxv = jnp.full((16,), qx)
        qyv = jnp.full((16,), qy)
        qzv = jnp.full((16,), qz)
        bqx = _rne_bf16(qxv)
        bqy = _rne_bf16(qyv)
        bqz = _rne_bf16(qzv)
        qnv = (qxv * qxv + qzv * qzv) + qyv * qyv
        sqv = (jnp.abs(qxv - bqx) + jnp.abs(qyv - bqy)
               + jnp.abs(qzv - bqz))
        e_q = e_base + 2.0 * sqv[0]
        cx = jnp.clip((qx * np.float32(_C)).astype(jnp.int32), 0, _C - 1)
        cy = jnp.clip((qy * np.float32(_C)).astype(jnp.int32), 0, _C - 1)
        cz = jnp.clip((qz * np.float32(_C)).astype(jnp.int32), 0, _C - 1)

        def _scan_range(s, e, keys, vals):
            ntrip = (e - s + 15) >> 4

            def _inner_body(it, st):
                keys, vals = st
                j = s + it * 16
                lanes = j + iota
                inb = lanes < e
                lc = jnp.minimum(lanes, e - 1)
                px = plsc.load_gather(px_v, [lc])
                py = plsc.load_gather(py_v, [lc])
                pz = plsc.load_gather(pz_v, [lc])
                pn = plsc.load_gather(pn_v, [lc])
                p0 = bqx * _rne_bf16(px)
                p1 = bqy * _rne_bf16(py)
                p2 = bqz * _rne_bf16(pz)
                s1 = p0 + p1
                bb = s1 - p0
                er1 = (p0 - (s1 - bb)) + (p1 - bb)
                s2 = s1 + p2
                bb2 = s2 - s1
                er2 = (s1 - (s2 - bb2)) + (p2 - bb2)
                mm = s2 + (er1 + er2)
                d2 = (qnv + pn) - 2.0 * mm
                key = jnp.where(inb & (d2 <= _R2), d2, _INF)
                beats = plsc.all_reduce_population_count(
                    (key <= jnp.full((16,), _w10(keys))) & (key < _INF))

                def _mb(_, st2):
                    return _merge(st2[0], st2[1], key, lc)

                keys, vals = lax.fori_loop(
                    0, jnp.minimum(beats[0], 1), _mb, (keys, vals))
                return keys, vals

            return lax.fori_loop(0, ntrip, _inner_body, (keys, vals))

        colc0 = (cx * _C + cy) * _C
        zs0 = jnp.maximum(cz - 1, 0)
        zs1 = jnp.minimum(cz + 1, _C - 1)
        zsel0 = colc0 + jnp.where(iota < 1, zs0, zs1 + 1)
        sev0 = plsc.load_gather(starts_v, [zsel0])
        seedk, _sv = _scan_range(sev0[0], sev0[1],
                                 jnp.full((16,), _INF),
                                 jnp.zeros((16,), jnp.int32))
        w10_cap = seedk[9]

        def _col_body(i, st):
            keys, vals = st
            dx = _sload(cdx_v, i)
            dy = _sload(cdy_v, i)
            m2 = _sload(cm2_v, i)
            ix = cx + dx
            iy = cy + dy
            okc = (ix >= 0) & (ix < _C) & (iy >= 0) & (iy < _C)
            ixc = jnp.clip(ix, 0, _C - 1)
            iyc = jnp.clip(iy, 0, _C - 1)
            thr = jnp.minimum(jnp.minimum(_w10(keys), w10_cap), _R2) + e_q
            active = okc & (_sload(clb2_v, i) <= thr)
            tc = (thr * np.float32(256.0)).astype(jnp.int32) + 1
            rem = jnp.clip(tc - m2, 0, 23)
            rz = _sload(lut_v, rem) + 1
            z0 = jnp.maximum(cz - rz, 0)
            z1 = jnp.minimum(cz + rz, _C - 1)
            colbase = (ixc * _C + iyc) * _C
            zsel = colbase + jnp.where(iota < 1, z0, z1 + 1)
            sev = plsc.load_gather(starts_v, [zsel])
            s = sev[0]
            e = jnp.where(active, sev[1], s)
            keys, vals = _scan_range(s, e, keys, vals)
            return keys, vals

        keys0 = jnp.full((16,), _INF)
        vals0 = jnp.zeros((16,), jnp.int32)
        keys, vals = lax.fori_loop(0, 9, _col_body, (keys0, vals0))
        thr_b = jnp.minimum(jnp.minimum(_w10(keys), w10_cap), _R2) + e_q
        tc_b = jnp.clip((thr_b * np.float32(256.0)).astype(jnp.int32) + 1,
                        0, 23)
        n_act = _sload(ccnt_v, tc_b)
        keys, vals = lax.fori_loop(9, n_act, _col_body, (keys, vals))

        shifted = jnp.take(keys, jnp.minimum(iota + 1, 15))
        tiec = plsc.all_reduce_population_count(
            (keys == shifted) & (iota < 15) & (shifted < _INF))

        def _fix(_, vv):
            oidx0 = plsc.load_gather(pidx_v, [vv])
            r = jnp.zeros((16,), jnp.int32)
            for k in range(16):
                kv = jnp.take(keys, jnp.full((16,), k, jnp.int32))
                r = r + (kv < keys).astype(jnp.int32)
            surrogate = (r << 14) | oidx0
            sout = plsc.sort_key_val(surrogate, vv)
            return sout[1]

        vals = lax.fori_loop(0, jnp.minimum(tiec[0], 1), _fix, vals)

        oidx = plsc.load_gather(pidx_v, [vals])
        pxo = plsc.load_gather(px_v, [vals])
        pyo = plsc.load_gather(py_v, [vals])
        pzo = plsc.load_gather(pz_v, [vals])
        valid = (keys <= _R2) & rank_mask
        sl = pl.ds(ql * 16, 16)
        oi_v[sl] = jnp.where(valid, oidx, 0)
        ox_v[sl] = jnp.where(valid, pxo, np.float32(0.0))
        oy_v[sl] = jnp.where(valid, pyo, np.float32(0.0))
        oz_v[sl] = jnp.where(valid, pzo, np.float32(0.0))

    for half in range(2):
        def _qstep(ql, _c, half=half):
            _do_query(half * _HALF + ql, ql)
            return _c

        lax.fori_loop(0, _HALF, _qstep, 0)
        off = (qbase + half * _HALF) * 16
        sz = _HALF * 16
        pltpu.sync_copy(oi_v, omap_h.at[pl.ds(off, sz)])
        pltpu.sync_copy(ox_v, ox_h.at[pl.ds(off, sz)])
        pltpu.sync_copy(oy_v, oy_h.at[pl.ds(off, sz)])
        pltpu.sync_copy(oz_v, oz_h.at[pl.ds(off, sz)])


_mesh = plsc.VectorSubcoreMesh(core_axis_name="c", subcore_axis_name="s")

_sc_call = pl.kernel(
    _sc_body,
    out_type=[
        jax.ShapeDtypeStruct((_NQ * 16,), jnp.int32),
        jax.ShapeDtypeStruct((_NQ * 16,), jnp.float32),
        jax.ShapeDtypeStruct((_NQ * 16,), jnp.float32),
        jax.ShapeDtypeStruct((_NQ * 16,), jnp.float32),
    ],
    mesh=_mesh,
    compiler_params=pltpu.CompilerParams(use_tc_tiling_on_sc=False, needs_layout_passes=False),
    scratch_types=[
        pltpu.VMEM((_NP,), jnp.float32),
        pltpu.VMEM((_NP,), jnp.float32),
        pltpu.VMEM((_NP,), jnp.float32),
        pltpu.VMEM((_NP,), jnp.float32),
        pltpu.VMEM((_NP,), jnp.int32),
        pltpu.VMEM((_NCELL + 24,), jnp.int32),
        pltpu.VMEM((_QPW + 16,), jnp.float32),
        pltpu.VMEM((_QPW + 16,), jnp.float32),
        pltpu.VMEM((_QPW + 16,), jnp.float32),
        pltpu.VMEM((_NCOLP + 16,), jnp.int32),
        pltpu.VMEM((_NCOLP + 16,), jnp.int32),
        pltpu.VMEM((_NCOLP + 16,), jnp.int32),
        pltpu.VMEM((_NCOLP + 16,), jnp.float32),
        pltpu.VMEM((40,), jnp.int32),
        pltpu.VMEM((40,), jnp.int32),
        pltpu.VMEM((_HALF * 16,), jnp.int32),
        pltpu.VMEM((_HALF * 16,), jnp.float32),
        pltpu.VMEM((_HALF * 16,), jnp.float32),
        pltpu.VMEM((_HALF * 16,), jnp.float32),
    ],
)


@jax.jit
def kernel(x, p_grid):
    pts = x[0]
    ci = jnp.clip(jnp.floor(pts * np.float32(_C)).astype(jnp.int32),
                  0, _C - 1)
    cid = (ci[:, 0] * _C + ci[:, 1]) * _C + ci[:, 2]
    order = jnp.argsort(cid).astype(jnp.int32)
    sp = jnp.take(pts, order, axis=0)
    cid_s = jnp.take(cid, order)
    starts = jnp.searchsorted(
        cid_s, jnp.arange(_NCELL + 1, dtype=jnp.int32)).astype(jnp.int32)
    starts = jnp.concatenate(
        [starts, jnp.full((7,), _NP, jnp.int32)])
    q = p_grid.reshape(-1, 3)

    omap, ox, oy, oz = _sc_call(
        jnp.copy(sp[:, 0]), jnp.copy(sp[:, 1]),
        jnp.copy(sp[:, 2]), order, starts,
        jnp.copy(q[:, 0]), jnp.copy(q[:, 1]),
        jnp.copy(q[:, 2]),
        jnp.asarray(_CDX), jnp.asarray(_CDY), jnp.asarray(_CM2),
        jnp.asarray(_CLB2), jnp.asarray(_ZLUT), jnp.asarray(_CCNT))

    mapping = omap.reshape(_NQ, 16)[:, :_K][None]
    outputs = jnp.stack(
        [ox.reshape(_NQ, 16)[:, :_K], oy.reshape(_NQ, 16)[:, :_K],
         oz.reshape(_NQ, 16)[:, :_K]], axis=-1)[None]
    return (mapping, outputs)


# speedup vs baseline: 1.2354x; 1.2354x over previous
import functools

import jax
import jax.numpy as jnp
import numpy as np
from jax import lax
from jax.experimental import pallas as pl
from jax.experimental.pallas import tpu as pltpu
from jax.experimental.pallas import tpu_sc as plsc

_C = 16
_NCELL = _C * _C * _C
_NP = 16384
_NQ = 32768
_K = 10
_R2 = np.float32(0.0625)
_INF = np.float32(np.inf)
_CELL2 = np.float32(1.0 / (_C * _C * _C * _C))
_NW = 32
_QPW = _NQ // _NW
_HALF = _QPW // 2

_cols = []
for _dx in range(-5, 6):
    for _dy in range(-5, 6):
        _m1 = max(abs(_dx) - 1, 0)
        _m2 = max(abs(_dy) - 1, 0)
        _mm = _m1 * _m1 + _m2 * _m2
        if _mm <= 22:
            _cols.append((_mm, _dx, _dy))
_cols.sort()
_NCOL = len(_cols)
_NCOLP = ((_NCOL + 7) // 8) * 8
_CDX = np.array([c[1] for c in _cols] + [0] * (_NCOLP - _NCOL), np.int32)
_CDY = np.array([c[2] for c in _cols] + [0] * (_NCOLP - _NCOL), np.int32)
_CM2 = np.array([c[0] for c in _cols] + [0] * (_NCOLP - _NCOL), np.int32)
_CLB2 = np.array(
    [c[0] * float(_CELL2) for c in _cols] + [np.inf] * (_NCOLP - _NCOL),
    np.float32)
_ZLUT = np.array([int(np.floor(np.sqrt(r))) for r in range(24)], np.int32)
_CCNT = np.array([sum(1 for c in _cols if c[0] <= t) for t in range(24)],
                 np.int32)

_IOTA = None


def _sload(ref, i):
    return ref[pl.ds(i, 16)][0]


def _rne_bf16(v):
    b = lax.bitcast_convert_type(v, jnp.uint32)
    r = (b + jnp.uint32(0x7FFF) + ((b >> jnp.uint32(16)) & jnp.uint32(1)))
    r = r & jnp.uint32(0xFFFF0000)
    return lax.bitcast_convert_type(r, jnp.float32)


def _sc_body(spx_h, spy_h, spz_h, pidx_h, starts_h, qx_h, qy_h, qz_h,
             cdx_h, cdy_h, cm2_h, clb2_h, lut_h, ccnt_h,
             omap_h, ox_h, oy_h, oz_h,
             px_v, py_v, pz_v, pn_v, pidx_v, starts_v,
             qx_v, qy_v, qz_v,
             cdx_v, cdy_v, cm2_v, clb2_v, lut_v, ccnt_v,
             oi_v, ox_v, oy_v, oz_v):
    wid = lax.axis_index("s") * 2 + lax.axis_index("c")
    qbase = wid * _QPW

    pltpu.sync_copy(spx_h, px_v)
    pltpu.sync_copy(spy_h, py_v)
    pltpu.sync_copy(spz_h, pz_v)
    pltpu.sync_copy(pidx_h, pidx_v)
    pltpu.sync_copy(starts_h, starts_v.at[pl.ds(0, _NCELL + 8)])
    pltpu.sync_copy(qx_h.at[pl.ds(qbase, _QPW)], qx_v.at[pl.ds(0, _QPW)])
    pltpu.sync_copy(qy_h.at[pl.ds(qbase, _QPW)], qy_v.at[pl.ds(0, _QPW)])
    pltpu.sync_copy(qz_h.at[pl.ds(qbase, _QPW)], qz_v.at[pl.ds(0, _QPW)])
    pltpu.sync_copy(cdx_h, cdx_v.at[pl.ds(0, _NCOLP)])
    pltpu.sync_copy(cdy_h, cdy_v.at[pl.ds(0, _NCOLP)])
    pltpu.sync_copy(cm2_h, cm2_v.at[pl.ds(0, _NCOLP)])
    pltpu.sync_copy(clb2_h, clb2_v.at[pl.ds(0, _NCOLP)])
    pltpu.sync_copy(lut_h, lut_v.at[pl.ds(0, 24)])
    pltpu.sync_copy(ccnt_h, ccnt_v.at[pl.ds(0, 24)])

    iota = lax.iota(jnp.int32, 16)
    rank_mask = iota < _K

    def _pn_step(i, smax):
        sl = pl.ds(i * 16, 16)
        px = px_v[sl]
        py = py_v[sl]
        pz = pz_v[sl]
        pn_v[sl] = (px * px + pz * pz) + py * py
        sp = (jnp.abs(px - _rne_bf16(px)) + jnp.abs(py - _rne_bf16(py))
              + jnp.abs(pz - _rne_bf16(pz)))
        return jnp.maximum(smax, sp)

    smax_vec = lax.fori_loop(0, _NP // 16, _pn_step,
                             jnp.zeros((16,), jnp.float32))
    for _sh in (8, 4, 2, 1):
        smax_vec = jnp.maximum(smax_vec, jnp.take(smax_vec, iota ^ _sh))
    e_base = 2.0 * smax_vec[0] + np.float32(1e-6)

    def _merge(keys, vals, nk, nv):
        sk, sv = plsc.sort_key_val(nk, nv)
        rk = lax.rev(sk, (0,))
        rv = lax.rev(sv, (0,))
        take = keys <= rk
        mk = jnp.where(take, keys, rk)
        mv = jnp.where(take, vals, rv)
        out = plsc.sort_key_val(mk, mv)
        return out[0], out[1]

    def _w10(keys):
        return keys[9]

    def _do_query(qi, ql):
        qx = _sload(qx_v, qi)
        qy = _sload(qy_v, qi)
        qz = _sload(qz_v, qi)
        qxv = jnp.full((16,), qx)
        qyv = jnp.full((16,), qy)
        qzv = jnp.full((16,), qz)
        bqx = _rne_bf16(qxv)
        bqy = _rne_bf16(qyv)
        bqz = _rne_bf16(qzv)
        qnv = (qxv * qxv + qzv * qzv) + qyv * qyv
        sqv = (jnp.abs(qxv - bqx) + jnp.abs(qyv - bqy)
               + jnp.abs(qzv - bqz))
        e_q = e_base + 2.0 * sqv[0]
        cx = jnp.clip((qx * np.float32(_C)).astype(jnp.int32), 0, _C - 1)
        cy = jnp.clip((qy * np.float32(_C)).astype(jnp.int32), 0, _C - 1)
        cz = jnp.clip((qz * np.float32(_C)).astype(jnp.int32), 0, _C - 1)

        def _scan_range(s, e, keys, vals):
            ntrip = (e - s + 15) >> 4

            def _inner_body(it, st):
                keys, vals = st
                j = s + it * 16
                lanes = j + iota
                inb = lanes < e
                lc = jnp.minimum(lanes, e - 1)
                px = plsc.load_gather(px_v, [lc])
                py = plsc.load_gather(py_v, [lc])
                pz = plsc.load_gather(pz_v, [lc])
                pn = plsc.load_gather(pn_v, [lc])
                p0 = bqx * _rne_bf16(px)
                p1 = bqy * _rne_bf16(py)
                p2 = bqz * _rne_bf16(pz)
                s1 = p0 + p1
                bb = s1 - p0
                er1 = (p0 - (s1 - bb)) + (p1 - bb)
                s2 = s1 + p2
                bb2 = s2 - s1
                er2 = (s1 - (s2 - bb2)) + (p2 - bb2)
                mm = s2 + (er1 + er2)
                d2 = (qnv + pn) - 2.0 * mm
                key = jnp.where(inb & (d2 <= _R2), d2, _INF)
                beats = plsc.all_reduce_population_count(
                    (key <= jnp.full((16,), _w10(keys))) & (key < _INF))

                def _mb(_, st2):
                    return _merge(st2[0], st2[1], key, lc)

                keys, vals = lax.fori_loop(
                    0, jnp.minimum(beats[0], 1), _mb, (keys, vals))
                return keys, vals

            return lax.fori_loop(0, ntrip, _inner_body, (keys, vals))

        colc0 = (cx * _C + cy) * _C
        zs0 = jnp.maximum(cz - 1, 0)
        zs1 = jnp.minimum(cz + 1, _C - 1)
        zsel0 = colc0 + jnp.where(iota < 1, zs0, zs1 + 1)
        sev0 = plsc.load_gather(starts_v, [zsel0])
        seedk, _sv = _scan_range(sev0[0], sev0[1],
                                 jnp.full((16,), _INF),
                                 jnp.zeros((16,), jnp.int32))
        w10_cap = seedk[9]

        def _grp_body(g, st):
            keys, vals = st
            base = g * 16
            dxv = cdx_v[pl.ds(base, 16)]
            dyv = cdy_v[pl.ds(base, 16)]
            m2v = cm2_v[pl.ds(base, 16)]
            lbv = clb2_v[pl.ds(base, 16)]
            ixv = cx + dxv
            iyv = cy + dyv
            okv = (ixv >= 0) & (ixv < _C) & (iyv >= 0) & (iyv < _C)
            thr = jnp.minimum(jnp.minimum(_w10(keys), w10_cap), _R2) + e_q
            actv = okv & (lbv <= thr)
            tcs = jnp.clip((thr * np.float32(256.0)).astype(jnp.int32) + 1,
                           0, 23)
            remv = jnp.clip(tcs - m2v, 0, 23)
            rzv = plsc.load_gather(lut_v, [remv]) + 1
            z0v = jnp.maximum(cz - rzv, 0)
            z1v = jnp.minimum(cz + rzv, _C - 1)
            cbv = (jnp.clip(ixv, 0, _C - 1) * _C
                   + jnp.clip(iyv, 0, _C - 1)) * _C
            sv = plsc.load_gather(starts_v, [cbv + z0v])
            ev = plsc.load_gather(starts_v, [cbv + z1v + 1])
            ev = jnp.where(actv, ev, sv)
            lenv = ev - sv
            ml = lenv
            for _sh in (8, 4, 2, 1):
                ml = jnp.maximum(ml, jnp.take(ml, iota ^ _sh))
            maxlen = ml[0]

            def _t_body(t, st2):
                keys, vals = st2
                tv = jnp.full((16,), t, jnp.int32)
                inb = tv < lenv
                lc = jnp.minimum(
                    sv + jnp.minimum(tv, jnp.maximum(lenv - 1, 0)),
                    _NP - 1)
                px = plsc.load_gather(px_v, [lc])
                py = plsc.load_gather(py_v, [lc])
                pz = plsc.load_gather(pz_v, [lc])
                pn = plsc.load_gather(pn_v, [lc])
                p0 = bqx * _rne_bf16(px)
                p1 = bqy * _rne_bf16(py)
                p2 = bqz * _rne_bf16(pz)
                s1 = p0 + p1
                bb = s1 - p0
                er1 = (p0 - (s1 - bb)) + (p1 - bb)
                s2 = s1 + p2
                bb2 = s2 - s1
                er2 = (s1 - (s2 - bb2)) + (p2 - bb2)
                mm = s2 + (er1 + er2)
                d2 = (qnv + pn) - 2.0 * mm
                key = jnp.where(inb & (d2 <= _R2), d2, _INF)
                beats = plsc.all_reduce_population_count(
                    (key <= jnp.full((16,), _w10(keys))) & (key < _INF))

                def _mb(_, st3):
                    return _merge(st3[0], st3[1], key, lc)

                keys, vals = lax.fori_loop(
                    0, jnp.minimum(beats[0], 1), _mb, (keys, vals))
                return keys, vals

            return lax.fori_loop(0, maxlen, _t_body, (keys, vals))

        keys0 = jnp.full((16,), _INF)
        vals0 = jnp.zeros((16,), jnp.int32)
        thr0 = jnp.minimum(w10_cap, _R2) + e_q
        tc0 = jnp.clip((thr0 * np.float32(256.0)).astype(jnp.int32) + 1,
                       0, 23)
        n_act = _sload(ccnt_v, tc0)
        n_grp = (n_act + 15) >> 4
        keys, vals = lax.fori_loop(0, n_grp, _grp_body, (keys0, vals0))

        shifted = jnp.take(keys, jnp.minimum(iota + 1, 15))
        tiec = plsc.all_reduce_population_count(
            (keys == shifted) & (iota < 15) & (shifted < _INF))

        def _fix(_, vv):
            oidx0 = plsc.load_gather(pidx_v, [vv])
            r = jnp.zeros((16,), jnp.int32)
            for k in range(16):
                kv = jnp.take(keys, jnp.full((16,), k, jnp.int32))
                r = r + (kv < keys).astype(jnp.int32)
            surrogate = (r << 14) | oidx0
            sout = plsc.sort_key_val(surrogate, vv)
            return sout[1]

        vals = lax.fori_loop(0, jnp.minimum(tiec[0], 1), _fix, vals)

        oidx = plsc.load_gather(pidx_v, [vals])
        pxo = plsc.load_gather(px_v, [vals])
        pyo = plsc.load_gather(py_v, [vals])
        pzo = plsc.load_gather(pz_v, [vals])
        valid = (keys <= _R2) & rank_mask
        sl = pl.ds(ql * 16, 16)
        oi_v[sl] = jnp.where(valid, oidx, 0)
        ox_v[sl] = jnp.where(valid, pxo, np.float32(0.0))
        oy_v[sl] = jnp.where(valid, pyo, np.float32(0.0))
        oz_v[sl] = jnp.where(valid, pzo, np.float32(0.0))

    for half in range(2):
        def _qstep(ql, _c, half=half):
            _do_query(half * _HALF + ql, ql)
            return _c

        lax.fori_loop(0, _HALF, _qstep, 0)
        off = (qbase + half * _HALF) * 16
        sz = _HALF * 16
        pltpu.sync_copy(oi_v, omap_h.at[pl.ds(off, sz)])
        pltpu.sync_copy(ox_v, ox_h.at[pl.ds(off, sz)])
        pltpu.sync_copy(oy_v, oy_h.at[pl.ds(off, sz)])
        pltpu.sync_copy(oz_v, oz_h.at[pl.ds(off, sz)])


_mesh = plsc.VectorSubcoreMesh(core_axis_name="c", subcore_axis_name="s")

_sc_call = pl.kernel(
    _sc_body,
    out_type=[
        jax.ShapeDtypeStruct((_NQ * 16,), jnp.int32),
        jax.ShapeDtypeStruct((_NQ * 16,), jnp.float32),
        jax.ShapeDtypeStruct((_NQ * 16,), jnp.float32),
        jax.ShapeDtypeStruct((_NQ * 16,), jnp.float32),
    ],
    mesh=_mesh,
    compiler_params=pltpu.CompilerParams(use_tc_tiling_on_sc=False, needs_layout_passes=False),
    scratch_types=[
        pltpu.VMEM((_NP,), jnp.float32),
        pltpu.VMEM((_NP,), jnp.float32),
        pltpu.VMEM((_NP,), jnp.float32),
        pltpu.VMEM((_NP,), jnp.float32),
        pltpu.VMEM((_NP,), jnp.int32),
        pltpu.VMEM((_NCELL + 24,), jnp.int32),
        pltpu.VMEM((_QPW + 16,), jnp.float32),
        pltpu.VMEM((_QPW + 16,), jnp.float32),
        pltpu.VMEM((_QPW + 16,), jnp.float32),
        pltpu.VMEM((_NCOLP + 16,), jnp.int32),
        pltpu.VMEM((_NCOLP + 16,), jnp.int32),
        pltpu.VMEM((_NCOLP + 16,), jnp.int32),
        pltpu.VMEM((_NCOLP + 16,), jnp.float32),
        pltpu.VMEM((40,), jnp.int32),
        pltpu.VMEM((40,), jnp.int32),
        pltpu.VMEM((_HALF * 16,), jnp.int32),
        pltpu.VMEM((_HALF * 16,), jnp.float32),
        pltpu.VMEM((_HALF * 16,), jnp.float32),
        pltpu.VMEM((_HALF * 16,), jnp.float32),
    ],
)


@jax.jit
def kernel(x, p_grid):
    pts = x[0]
    ci = jnp.clip(jnp.floor(pts * np.float32(_C)).astype(jnp.int32),
                  0, _C - 1)
    cid = (ci[:, 0] * _C + ci[:, 1]) * _C + ci[:, 2]
    order = jnp.argsort(cid).astype(jnp.int32)
    sp = jnp.take(pts, order, axis=0)
    cid_s = jnp.take(cid, order)
    starts = jnp.searchsorted(
        cid_s, jnp.arange(_NCELL + 1, dtype=jnp.int32)).astype(jnp.int32)
    starts = jnp.concatenate(
        [starts, jnp.full((7,), _NP, jnp.int32)])
    q = p_grid.reshape(-1, 3)

    omap, ox, oy, oz = _sc_call(
        jnp.copy(sp[:, 0]), jnp.copy(sp[:, 1]),
        jnp.copy(sp[:, 2]), order, starts,
        jnp.copy(q[:, 0]), jnp.copy(q[:, 1]),
        jnp.copy(q[:, 2]),
        jnp.asarray(_CDX), jnp.asarray(_CDY), jnp.asarray(_CM2),
        jnp.asarray(_CLB2), jnp.asarray(_ZLUT), jnp.asarray(_CCNT))

    mapping = omap.reshape(_NQ, 16)[:, :_K][None]
    outputs = jnp.stack(
        [ox.reshape(_NQ, 16)[:, :_K], oy.reshape(_NQ, 16)[:, :_K],
         oz.reshape(_NQ, 16)[:, :_K]], axis=-1)[None]
    return (mapping, outputs)


# speedup vs baseline: 1.9029x; 1.5403x over previous
import functools

import jax
import jax.numpy as jnp
import numpy as np
from jax import lax
from jax.experimental import pallas as pl
from jax.experimental.pallas import tpu as pltpu
from jax.experimental.pallas import tpu_sc as plsc

_C = 16
_NCELL = _C * _C * _C
_NP = 16384
_NQ = 32768
_K = 10
_R2 = np.float32(0.0625)
_INF = np.float32(np.inf)
_CELL2 = np.float32(1.0 / (_C * _C * _C * _C))
_NW = 32
_QPW = _NQ // _NW
_HALF = _QPW // 2

_cols = []
for _dx in range(-5, 6):
    for _dy in range(-5, 6):
        _m1 = max(abs(_dx) - 1, 0)
        _m2 = max(abs(_dy) - 1, 0)
        _mm = _m1 * _m1 + _m2 * _m2
        if _mm <= 22:
            _cols.append((_mm, _dx, _dy))
_cols.sort()
_NCOL = len(_cols)
_NCOLP = ((_NCOL + 7) // 8) * 8
_CDX = np.array([c[1] for c in _cols] + [0] * (_NCOLP - _NCOL), np.int32)
_CDY = np.array([c[2] for c in _cols] + [0] * (_NCOLP - _NCOL), np.int32)
_CM2 = np.array([c[0] for c in _cols] + [0] * (_NCOLP - _NCOL), np.int32)
_CLB2 = np.array(
    [c[0] * float(_CELL2) for c in _cols] + [np.inf] * (_NCOLP - _NCOL),
    np.float32)
_ZLUT = np.array([int(np.floor(np.sqrt(r))) for r in range(24)], np.int32)
_CCNT = np.array([sum(1 for c in _cols if c[0] <= t) for t in range(24)],
                 np.int32)

_IOTA = None


def _sload(ref, i):
    return ref[pl.ds(i, 16)][0]


def _rne_bf16(v):
    b = lax.bitcast_convert_type(v, jnp.uint32)
    r = (b + jnp.uint32(0x7FFF) + ((b >> jnp.uint32(16)) & jnp.uint32(1)))
    r = r & jnp.uint32(0xFFFF0000)
    return lax.bitcast_convert_type(r, jnp.float32)


def _sc_body(spx_h, spy_h, spz_h, pidx_h, starts_h, qx_h, qy_h, qz_h,
             cdx_h, cdy_h, cm2_h, clb2_h, lut_h, ccnt_h,
             omap_h, ox_h, oy_h, oz_h,
             px_v, py_v, pz_v, pn_v, pidx_v, starts_v,
             qx_v, qy_v, qz_v,
             cdx_v, cdy_v, cm2_v, clb2_v, lut_v, ccnt_v,
             oi_v, ox_v, oy_v, oz_v):
    wid = lax.axis_index("s") * 2 + lax.axis_index("c")
    qbase = wid * _QPW

    pltpu.sync_copy(spx_h, px_v)
    pltpu.sync_copy(spy_h, py_v)
    pltpu.sync_copy(spz_h, pz_v)
    pltpu.sync_copy(pidx_h, pidx_v)
    pltpu.sync_copy(starts_h, starts_v.at[pl.ds(0, _NCELL + 8)])
    pltpu.sync_copy(qx_h.at[pl.ds(qbase, _QPW)], qx_v.at[pl.ds(0, _QPW)])
    pltpu.sync_copy(qy_h.at[pl.ds(qbase, _QPW)], qy_v.at[pl.ds(0, _QPW)])
    pltpu.sync_copy(qz_h.at[pl.ds(qbase, _QPW)], qz_v.at[pl.ds(0, _QPW)])
    pltpu.sync_copy(cdx_h, cdx_v.at[pl.ds(0, _NCOLP)])
    pltpu.sync_copy(cdy_h, cdy_v.at[pl.ds(0, _NCOLP)])
    pltpu.sync_copy(cm2_h, cm2_v.at[pl.ds(0, _NCOLP)])
    pltpu.sync_copy(clb2_h, clb2_v.at[pl.ds(0, _NCOLP)])
    pltpu.sync_copy(lut_h, lut_v.at[pl.ds(0, 24)])
    pltpu.sync_copy(ccnt_h, ccnt_v.at[pl.ds(0, 24)])

    iota = lax.iota(jnp.int32, 16)
    rank_mask = iota < _K

    def _pn_step(i, smax):
        sl = pl.ds(i * 16, 16)
        px = px_v[sl]
        py = py_v[sl]
        pz = pz_v[sl]
        pn_v[sl] = (px * px + pz * pz) + py * py
        sp = (jnp.abs(px - _rne_bf16(px)) + jnp.abs(py - _rne_bf16(py))
              + jnp.abs(pz - _rne_bf16(pz)))
        return jnp.maximum(smax, sp)

    smax_vec = lax.fori_loop(0, _NP // 16, _pn_step,
                             jnp.zeros((16,), jnp.float32))
    for _sh in (8, 4, 2, 1):
        smax_vec = jnp.maximum(smax_vec, jnp.take(smax_vec, iota ^ _sh))
    e_base = 2.0 * smax_vec[0] + np.float32(1e-6)

    def _merge(keys, vals, nk, nv):
        sk, sv = plsc.sort_key_val(nk, nv)
        rk = lax.rev(sk, (0,))
        rv = lax.rev(sv, (0,))
        take = keys <= rk
        mk = jnp.where(take, keys, rk)
        mv = jnp.where(take, vals, rv)
        out = plsc.sort_key_val(mk, mv)
        return out[0], out[1]

    def _w10(keys):
        return keys[9]

    def _do_query(qi, ql):
        qx = _sload(qx_v, qi)
        qy = _sload(qy_v, qi)
        qz = _sload(qz_v, qi)
        qxv = jnp.full((16,), qx)
        qyv = jnp.full((16,), qy)
        qzv = jnp.full((16,), qz)
        bqx = _rne_bf16(qxv)
        bqy = _rne_bf16(qyv)
        bqz = _rne_bf16(qzv)
        qnv = (qxv * qxv + qzv * qzv) + qyv * qyv
        sqv = (jnp.abs(qxv - bqx) + jnp.abs(qyv - bqy)
               + jnp.abs(qzv - bqz))
        e_q = e_base + 2.0 * sqv[0]
        cx = jnp.clip((qx * np.float32(_C)).astype(jnp.int32), 0, _C - 1)
        cy = jnp.clip((qy * np.float32(_C)).astype(jnp.int32), 0, _C - 1)
        cz = jnp.clip((qz * np.float32(_C)).astype(jnp.int32), 0, _C - 1)

        def _scan_range(s, e, keys, vals):
            ntrip = (e - s + 15) >> 4

            def _inner_body(it, st):
                keys, vals = st
                j = s + it * 16
                lanes = j + iota
                inb = lanes < e
                lc = jnp.minimum(lanes, e - 1)
                px = plsc.load_gather(px_v, [lc])
                py = plsc.load_gather(py_v, [lc])
                pz = plsc.load_gather(pz_v, [lc])
                pn = plsc.load_gather(pn_v, [lc])
                p0 = bqx * _rne_bf16(px)
                p1 = bqy * _rne_bf16(py)
                p2 = bqz * _rne_bf16(pz)
                s1 = p0 + p1
                bb = s1 - p0
                er1 = (p0 - (s1 - bb)) + (p1 - bb)
                s2 = s1 + p2
                bb2 = s2 - s1
                er2 = (s1 - (s2 - bb2)) + (p2 - bb2)
                mm = s2 + (er1 + er2)
                d2 = (qnv + pn) - 2.0 * mm
                key = jnp.where(inb & (d2 <= _R2), d2, _INF)
                beats = plsc.all_reduce_population_count(
                    (key <= jnp.full((16,), _w10(keys))) & (key < _INF))

                def _mb(_, st2):
                    return _merge(st2[0], st2[1], key, lc)

                keys, vals = lax.fori_loop(
                    0, jnp.minimum(beats[0], 1), _mb, (keys, vals))
                return keys, vals

            return lax.fori_loop(0, ntrip, _inner_body, (keys, vals))

        colc0 = (cx * _C + cy) * _C
        zs0 = jnp.maximum(cz - 1, 0)
        zs1 = jnp.minimum(cz + 1, _C - 1)
        zsel0 = colc0 + jnp.where(iota < 1, zs0, zs1 + 1)
        sev0 = plsc.load_gather(starts_v, [zsel0])
        seedk, _sv = _scan_range(sev0[0], sev0[1],
                                 jnp.full((16,), _INF),
                                 jnp.zeros((16,), jnp.int32))
        w10_cap = seedk[9]

        def _grp_body(g, st):
            keys, vals = st
            base = g * 16
            dxv = cdx_v[pl.ds(base, 16)]
            dyv = cdy_v[pl.ds(base, 16)]
            m2v = cm2_v[pl.ds(base, 16)]
            lbv = clb2_v[pl.ds(base, 16)]
            ixv = cx + dxv
            iyv = cy + dyv
            okv = (ixv >= 0) & (ixv < _C) & (iyv >= 0) & (iyv < _C)
            thr = jnp.minimum(jnp.minimum(_w10(keys), w10_cap), _R2) + e_q
            actv = okv & (lbv <= thr)
            tcs = jnp.clip((thr * np.float32(256.0)).astype(jnp.int32) + 1,
                           0, 23)
            remv = jnp.clip(tcs - m2v, 0, 23)
            rzv = plsc.load_gather(lut_v, [remv]) + 1
            z0v = jnp.maximum(cz - rzv, 0)
            z1v = jnp.minimum(cz + rzv, _C - 1)
            cbv = (jnp.clip(ixv, 0, _C - 1) * _C
                   + jnp.clip(iyv, 0, _C - 1)) * _C
            sv = plsc.load_gather(starts_v, [cbv + z0v])
            ev = plsc.load_gather(starts_v, [cbv + z1v + 1])
            ev = jnp.where(actv, ev, sv)
            lenv = ev - sv
            pref = plsc.cumsum(lenv)
            m_tot = pref[15]
            nb = (m_tot + 15) >> 4

            def _t_body(t, st2):
                keys, vals = st2
                flat = t * 16 + iota
                c = jnp.zeros((16,), jnp.int32)
                for _step in (8, 4, 2, 1):
                    pv = jnp.take(pref, c + (_step - 1))
                    c = c + jnp.where(pv <= flat, _step, 0)
                cc = jnp.minimum(c, 15)
                excl = jnp.where(c == 0, 0,
                                 jnp.take(pref, jnp.maximum(c - 1, 0)))
                inb = flat < jnp.full((16,), m_tot)
                lc = jnp.minimum(jnp.take(sv, cc) + (flat - excl), _NP - 1)
                px = plsc.load_gather(px_v, [lc])
                py = plsc.load_gather(py_v, [lc])
                pz = plsc.load_gather(pz_v, [lc])
                pn = plsc.load_gather(pn_v, [lc])
                p0 = bqx * _rne_bf16(px)
                p1 = bqy * _rne_bf16(py)
                p2 = bqz * _rne_bf16(pz)
                s1 = p0 + p1
                bb = s1 - p0
                er1 = (p0 - (s1 - bb)) + (p1 - bb)
                s2 = s1 + p2
                bb2 = s2 - s1
                er2 = (s1 - (s2 - bb2)) + (p2 - bb2)
                mm = s2 + (er1 + er2)
                d2 = (qnv + pn) - 2.0 * mm
                key = jnp.where(inb & (d2 <= _R2), d2, _INF)
                beats = plsc.all_reduce_population_count(
                    (key <= jnp.full((16,), _w10(keys))) & (key < _INF))

                def _mb(_, st3):
                    return _merge(st3[0], st3[1], key, lc)

                keys, vals = lax.fori_loop(
                    0, jnp.minimum(beats[0], 1), _mb, (keys, vals))
                return keys, vals

            return lax.fori_loop(0, nb, _t_body, (keys, vals))

        keys0 = jnp.full((16,), _INF)
        vals0 = jnp.zeros((16,), jnp.int32)
        thr0 = jnp.minimum(w10_cap, _R2) + e_q
        tc0 = jnp.clip((thr0 * np.float32(256.0)).astype(jnp.int32) + 1,
                       0, 23)
        n_act = _sload(ccnt_v, tc0)
        n_grp = (n_act + 15) >> 4
        keys, vals = lax.fori_loop(0, n_grp, _grp_body, (keys0, vals0))

        shifted = jnp.take(keys, jnp.minimum(iota + 1, 15))
        tiec = plsc.all_reduce_population_count(
            (keys == shifted) & (iota < 15) & (shifted < _INF))

        def _fix(_, vv):
            oidx0 = plsc.load_gather(pidx_v, [vv])
            r = jnp.zeros((16,), jnp.int32)
            for k in range(16):
                kv = jnp.take(keys, jnp.full((16,), k, jnp.int32))
                r = r + (kv < keys).astype(jnp.int32)
            surrogate = (r << 14) | oidx0
            sout = plsc.sort_key_val(surrogate, vv)
            return sout[1]

        vals = lax.fori_loop(0, jnp.minimum(tiec[0], 1), _fix, vals)

        oidx = plsc.load_gather(pidx_v, [vals])
        pxo = plsc.load_gather(px_v, [vals])
        pyo = plsc.load_gather(py_v, [vals])
        pzo = plsc.load_gather(pz_v, [vals])
        valid = (keys <= _R2) & rank_mask
        sl = pl.ds(ql * 16, 16)
        oi_v[sl] = jnp.where(valid, oidx, 0)
        ox_v[sl] = jnp.where(valid, pxo, np.float32(0.0))
        oy_v[sl] = jnp.where(valid, pyo, np.float32(0.0))
        oz_v[sl] = jnp.where(valid, pzo, np.float32(0.0))

    for half in range(2):
        def _qstep(ql, _c, half=half):
            _do_query(half * _HALF + ql, ql)
            return _c

        lax.fori_loop(0, _HALF, _qstep, 0)
        off = (qbase + half * _HALF) * 16
        sz = _HALF * 16
        pltpu.sync_copy(oi_v, omap_h.at[pl.ds(off, sz)])
        pltpu.sync_copy(ox_v, ox_h.at[pl.ds(off, sz)])
        pltpu.sync_copy(oy_v, oy_h.at[pl.ds(off, sz)])
        pltpu.sync_copy(oz_v, oz_h.at[pl.ds(off, sz)])


_mesh = plsc.VectorSubcoreMesh(core_axis_name="c", subcore_axis_name="s")

_sc_call = pl.kernel(
    _sc_body,
    out_type=[
        jax.ShapeDtypeStruct((_NQ * 16,), jnp.int32),
        jax.ShapeDtypeStruct((_NQ * 16,), jnp.float32),
        jax.ShapeDtypeStruct((_NQ * 16,), jnp.float32),
        jax.ShapeDtypeStruct((_NQ * 16,), jnp.float32),
    ],
    mesh=_mesh,
    compiler_params=pltpu.CompilerParams(use_tc_tiling_on_sc=False, needs_layout_passes=False),
    scratch_types=[
        pltpu.VMEM((_NP,), jnp.float32),
        pltpu.VMEM((_NP,), jnp.float32),
        pltpu.VMEM((_NP,), jnp.float32),
        pltpu.VMEM((_NP,), jnp.float32),
        pltpu.VMEM((_NP,), jnp.int32),
        pltpu.VMEM((_NCELL + 24,), jnp.int32),
        pltpu.VMEM((_QPW + 16,), jnp.float32),
        pltpu.VMEM((_QPW + 16,), jnp.float32),
        pltpu.VMEM((_QPW + 16,), jnp.float32),
        pltpu.VMEM((_NCOLP + 16,), jnp.int32),
        pltpu.VMEM((_NCOLP + 16,), jnp.int32),
        pltpu.VMEM((_NCOLP + 16,), jnp.int32),
        pltpu.VMEM((_NCOLP + 16,), jnp.float32),
        pltpu.VMEM((40,), jnp.int32),
        pltpu.VMEM((40,), jnp.int32),
        pltpu.VMEM((_HALF * 16,), jnp.int32),
        pltpu.VMEM((_HALF * 16,), jnp.float32),
        pltpu.VMEM((_HALF * 16,), jnp.float32),
        pltpu.VMEM((_HALF * 16,), jnp.float32),
    ],
)


@jax.jit
def kernel(x, p_grid):
    pts = x[0]
    ci = jnp.clip(jnp.floor(pts * np.float32(_C)).astype(jnp.int32),
                  0, _C - 1)
    cid = (ci[:, 0] * _C + ci[:, 1]) * _C + ci[:, 2]
    order = jnp.argsort(cid).astype(jnp.int32)
    sp = jnp.take(pts, order, axis=0)
    cid_s = jnp.take(cid, order)
    starts = jnp.searchsorted(
        cid_s, jnp.arange(_NCELL + 1, dtype=jnp.int32)).astype(jnp.int32)
    starts = jnp.concatenate(
        [starts, jnp.full((7,), _NP, jnp.int32)])
    q = p_grid.reshape(-1, 3)

    omap, ox, oy, oz = _sc_call(
        jnp.copy(sp[:, 0]), jnp.copy(sp[:, 1]),
        jnp.copy(sp[:, 2]), order, starts,
        jnp.copy(q[:, 0]), jnp.copy(q[:, 1]),
        jnp.copy(q[:, 2]),
        jnp.asarray(_CDX), jnp.asarray(_CDY), jnp.asarray(_CM2),
        jnp.asarray(_CLB2), jnp.asarray(_ZLUT), jnp.asarray(_CCNT))

    mapping = omap.reshape(_NQ, 16)[:, :_K][None]
    outputs = jnp.stack(
        [ox.reshape(_NQ, 16)[:, :_K], oy.reshape(_NQ, 16)[:, :_K],
         oz.reshape(_NQ, 16)[:, :_K]], axis=-1)[None]
    return (mapping, outputs)


# speedup vs baseline: 2.3108x; 1.2144x over previous
import functools

import jax
import jax.numpy as jnp
import numpy as np
from jax import lax
from jax.experimental import pallas as pl
from jax.experimental.pallas import tpu as pltpu
from jax.experimental.pallas import tpu_sc as plsc

_C = 16
_NCELL = _C * _C * _C
_NP = 16384
_NQ = 32768
_K = 10
_R2 = np.float32(0.0625)
_INF = np.float32(np.inf)
_CELL2 = np.float32(1.0 / (_C * _C * _C * _C))
_NW = 32
_QPW = _NQ // _NW
_HALF = _QPW // 2

_cols = []
for _dx in range(-5, 6):
    for _dy in range(-5, 6):
        _m1 = max(abs(_dx) - 1, 0)
        _m2 = max(abs(_dy) - 1, 0)
        _mm = _m1 * _m1 + _m2 * _m2
        if _mm <= 22:
            _cols.append((_mm, _dx, _dy))
_cols.sort()
_NCOL = len(_cols)
_NCOLP = ((_NCOL + 7) // 8) * 8
_CDX = np.array([c[1] for c in _cols] + [0] * (_NCOLP - _NCOL), np.int32)
_CDY = np.array([c[2] for c in _cols] + [0] * (_NCOLP - _NCOL), np.int32)
_CM2 = np.array([c[0] for c in _cols] + [0] * (_NCOLP - _NCOL), np.int32)
_CLB2 = np.array(
    [c[0] * float(_CELL2) for c in _cols] + [np.inf] * (_NCOLP - _NCOL),
    np.float32)
_ZLUT = np.array([int(np.floor(np.sqrt(r))) for r in range(24)], np.int32)
_CCNT = np.array([sum(1 for c in _cols if c[0] <= t) for t in range(24)],
                 np.int32)

_IOTA = None


def _sload(ref, i):
    return ref[pl.ds(i, 16)][0]


def _rne_bf16(v):
    b = lax.bitcast_convert_type(v, jnp.uint32)
    r = (b + jnp.uint32(0x7FFF) + ((b >> jnp.uint32(16)) & jnp.uint32(1)))
    r = r & jnp.uint32(0xFFFF0000)
    return lax.bitcast_convert_type(r, jnp.float32)


def _sc_body(spx_h, spy_h, spz_h, pidx_h, starts_h, qx_h, qy_h, qz_h,
             cdx_h, cdy_h, cm2_h, clb2_h, lut_h, ccnt_h,
             omap_h, ox_h, oy_h, oz_h,
             px_v, py_v, pz_v, pn_v, pidx_v, starts_v,
             qx_v, qy_v, qz_v,
             cdx_v, cdy_v, cm2_v, clb2_v, lut_v, ccnt_v,
             oi_v, ox_v, oy_v, oz_v):
    wid = lax.axis_index("s") * 2 + lax.axis_index("c")
    qbase = wid * _QPW

    pltpu.sync_copy(spx_h, px_v)
    pltpu.sync_copy(spy_h, py_v)
    pltpu.sync_copy(spz_h, pz_v)
    pltpu.sync_copy(pidx_h, pidx_v)
    pltpu.sync_copy(starts_h, starts_v.at[pl.ds(0, _NCELL + 8)])
    pltpu.sync_copy(qx_h.at[pl.ds(qbase, _QPW)], qx_v.at[pl.ds(0, _QPW)])
    pltpu.sync_copy(qy_h.at[pl.ds(qbase, _QPW)], qy_v.at[pl.ds(0, _QPW)])
    pltpu.sync_copy(qz_h.at[pl.ds(qbase, _QPW)], qz_v.at[pl.ds(0, _QPW)])
    pltpu.sync_copy(cdx_h, cdx_v.at[pl.ds(0, _NCOLP)])
    pltpu.sync_copy(cdy_h, cdy_v.at[pl.ds(0, _NCOLP)])
    pltpu.sync_copy(cm2_h, cm2_v.at[pl.ds(0, _NCOLP)])
    pltpu.sync_copy(clb2_h, clb2_v.at[pl.ds(0, _NCOLP)])
    pltpu.sync_copy(lut_h, lut_v.at[pl.ds(0, 24)])
    pltpu.sync_copy(ccnt_h, ccnt_v.at[pl.ds(0, 24)])

    iota = lax.iota(jnp.int32, 16)
    rank_mask = iota < _K

    def _pn_step(i, smax):
        sl = pl.ds(i * 16, 16)
        px = px_v[sl]
        py = py_v[sl]
        pz = pz_v[sl]
        pn_v[sl] = (px * px + pz * pz) + py * py
        sp = (jnp.abs(px - _rne_bf16(px)) + jnp.abs(py - _rne_bf16(py))
              + jnp.abs(pz - _rne_bf16(pz)))
        return jnp.maximum(smax, sp)

    smax_vec = lax.fori_loop(0, _NP // 16, _pn_step,
                             jnp.zeros((16,), jnp.float32))
    for _sh in (8, 4, 2, 1):
        smax_vec = jnp.maximum(smax_vec, jnp.take(smax_vec, iota ^ _sh))
    e_base = 2.0 * smax_vec[0] + np.float32(1e-6)

    def _merge(keys, vals, nk, nv):
        sk, sv = plsc.sort_key_val(nk, nv)
        rk = lax.rev(sk, (0,))
        rv = lax.rev(sv, (0,))
        take = keys <= rk
        mk = jnp.where(take, keys, rk)
        mv = jnp.where(take, vals, rv)
        out = plsc.sort_key_val(mk, mv)
        return out[0], out[1]

    def _w10(keys):
        return keys[9]

    def _do_query(qi, ql):
        qx = _sload(qx_v, qi)
        qy = _sload(qy_v, qi)
        qz = _sload(qz_v, qi)
        qxv = jnp.full((16,), qx)
        qyv = jnp.full((16,), qy)
        qzv = jnp.full((16,), qz)
        bqx = _rne_bf16(qxv)
        bqy = _rne_bf16(qyv)
        bqz = _rne_bf16(qzv)
        qnv = (qxv * qxv + qzv * qzv) + qyv * qyv
        sqv = (jnp.abs(qxv - bqx) + jnp.abs(qyv - bqy)
               + jnp.abs(qzv - bqz))
        e_q = e_base + 2.0 * sqv[0]
        cx = jnp.clip((qx * np.float32(_C)).astype(jnp.int32), 0, _C - 1)
        cy = jnp.clip((qy * np.float32(_C)).astype(jnp.int32), 0, _C - 1)
        cz = jnp.clip((qz * np.float32(_C)).astype(jnp.int32), 0, _C - 1)

        def _scan_range(s, e, keys, vals):
            ntrip = (e - s + 15) >> 4

            def _inner_body(it, st):
                keys, vals = st
                j = s + it * 16
                lanes = j + iota
                inb = lanes < e
                lc = jnp.minimum(lanes, e - 1)
                px = plsc.load_gather(px_v, [lc])
                py = plsc.load_gather(py_v, [lc])
                pz = plsc.load_gather(pz_v, [lc])
                pn = plsc.load_gather(pn_v, [lc])
                p0 = bqx * _rne_bf16(px)
                p1 = bqy * _rne_bf16(py)
                p2 = bqz * _rne_bf16(pz)
                s1 = p0 + p1
                bb = s1 - p0
                er1 = (p0 - (s1 - bb)) + (p1 - bb)
                s2 = s1 + p2
                bb2 = s2 - s1
                er2 = (s1 - (s2 - bb2)) + (p2 - bb2)
                mm = s2 + (er1 + er2)
                d2 = (qnv + pn) - 2.0 * mm
                key = jnp.where(inb & (d2 <= _R2), d2, _INF)
                beats = plsc.all_reduce_population_count(
                    (key <= jnp.full((16,), _w10(keys))) & (key < _INF))

                def _mb(_, st2):
                    return _merge(st2[0], st2[1], key, lc)

                keys, vals = lax.fori_loop(
                    0, jnp.minimum(beats[0], 1), _mb, (keys, vals))
                return keys, vals

            return lax.fori_loop(0, ntrip, _inner_body, (keys, vals))

        colc0 = (cx * _C + cy) * _C
        zs0 = jnp.maximum(cz - 1, 0)
        zs1 = jnp.minimum(cz + 1, _C - 1)
        zsel0 = colc0 + jnp.where(iota < 1, zs0, zs1 + 1)
        sev0 = plsc.load_gather(starts_v, [zsel0])
        seedk, _sv = _scan_range(sev0[0], sev0[1],
                                 jnp.full((16,), _INF),
                                 jnp.zeros((16,), jnp.int32))
        w10_cap = seedk[9]

        def _grp_body(g, st):
            keys, vals = st
            base = g * 16
            dxv = cdx_v[pl.ds(base, 16)]
            dyv = cdy_v[pl.ds(base, 16)]
            m2v = cm2_v[pl.ds(base, 16)]
            lbv = clb2_v[pl.ds(base, 16)]
            ixv = cx + dxv
            iyv = cy + dyv
            okv = (ixv >= 0) & (ixv < _C) & (iyv >= 0) & (iyv < _C)
            thr = jnp.minimum(jnp.minimum(_w10(keys), w10_cap), _R2) + e_q
            actv = okv & (lbv <= thr)
            tcs = jnp.clip((thr * np.float32(256.0)).astype(jnp.int32) + 1,
                           0, 23)
            remv = jnp.clip(tcs - m2v, 0, 23)
            rzv = plsc.load_gather(lut_v, [remv]) + 1
            z0v = jnp.maximum(cz - rzv, 0)
            z1v = jnp.minimum(cz + rzv, _C - 1)
            cbv = (jnp.clip(ixv, 0, _C - 1) * _C
                   + jnp.clip(iyv, 0, _C - 1)) * _C
            sv = plsc.load_gather(starts_v, [cbv + z0v])
            ev = plsc.load_gather(starts_v, [cbv + z1v + 1])
            ev = jnp.where(actv, ev, sv)
            lenv = ev - sv
            pref = plsc.cumsum(lenv)
            m_tot = pref[15]
            nb = (m_tot + 15) >> 4

            def _t_body(t, st2):
                keys, vals = st2
                flat = t * 16 + iota
                c = jnp.zeros((16,), jnp.int32)
                for _step in (8, 4, 2, 1):
                    pv = jnp.take(pref, c + (_step - 1))
                    c = c + jnp.where(pv <= flat, _step, 0)
                cc = jnp.minimum(c, 15)
                excl = jnp.where(c == 0, 0,
                                 jnp.take(pref, jnp.maximum(c - 1, 0)))
                inb = flat < jnp.full((16,), m_tot)
                lc = jnp.minimum(jnp.take(sv, cc) + (flat - excl), _NP - 1)
                px = plsc.load_gather(px_v, [lc])
                py = plsc.load_gather(py_v, [lc])
                pz = plsc.load_gather(pz_v, [lc])
                pn = plsc.load_gather(pn_v, [lc])
                p0 = bqx * _rne_bf16(px)
                p1 = bqy * _rne_bf16(py)
                p2 = bqz * _rne_bf16(pz)
                s1 = p0 + p1
                bb = s1 - p0
                er1 = (p0 - (s1 - bb)) + (p1 - bb)
                s2 = s1 + p2
                bb2 = s2 - s1
                er2 = (s1 - (s2 - bb2)) + (p2 - bb2)
                mm = s2 + (er1 + er2)
                d2 = (qnv + pn) - 2.0 * mm
                key = jnp.where(inb & (d2 <= _R2), d2, _INF)
                beats = plsc.all_reduce_population_count(
                    (key <= jnp.full((16,), _w10(keys))) & (key < _INF))

                def _mb(_, st3):
                    return _merge(st3[0], st3[1], key, lc)

                keys, vals = lax.fori_loop(
                    0, jnp.minimum(beats[0], 1), _mb, (keys, vals))
                return keys, vals

            return lax.fori_loop(0, nb, _t_body, (keys, vals))

        keys0 = jnp.full((16,), _INF)
        vals0 = jnp.zeros((16,), jnp.int32)
        keys, vals = _grp_body(0, (keys0, vals0))
        thr0 = jnp.minimum(jnp.minimum(_w10(keys), w10_cap), _R2) + e_q
        tc0 = jnp.clip((thr0 * np.float32(256.0)).astype(jnp.int32) + 1,
                       0, 23)
        n_act = _sload(ccnt_v, tc0)
        n_grp = (n_act + 15) >> 4
        keys, vals = lax.fori_loop(1, n_grp, _grp_body, (keys, vals))

        shifted = jnp.take(keys, jnp.minimum(iota + 1, 15))
        tiec = plsc.all_reduce_population_count(
            (keys == shifted) & (iota < 15) & (shifted < _INF))

        def _fix(_, vv):
            oidx0 = plsc.load_gather(pidx_v, [vv])
            r = jnp.zeros((16,), jnp.int32)
            for k in range(16):
                kv = jnp.take(keys, jnp.full((16,), k, jnp.int32))
                r = r + (kv < keys).astype(jnp.int32)
            surrogate = (r << 14) | oidx0
            sout = plsc.sort_key_val(surrogate, vv)
            return sout[1]

        vals = lax.fori_loop(0, jnp.minimum(tiec[0], 1), _fix, vals)

        oidx = plsc.load_gather(pidx_v, [vals])
        pxo = plsc.load_gather(px_v, [vals])
        pyo = plsc.load_gather(py_v, [vals])
        pzo = plsc.load_gather(pz_v, [vals])
        valid = (keys <= _R2) & rank_mask
        sl = pl.ds(ql * 16, 16)
        oi_v[sl] = jnp.where(valid, oidx, 0)
        ox_v[sl] = jnp.where(valid, pxo, np.float32(0.0))
        oy_v[sl] = jnp.where(valid, pyo, np.float32(0.0))
        oz_v[sl] = jnp.where(valid, pzo, np.float32(0.0))

    for half in range(2):
        def _qstep(ql, _c, half=half):
            _do_query(half * _HALF + ql, ql)
            return _c

        lax.fori_loop(0, _HALF, _qstep, 0)
        off = (qbase + half * _HALF) * 16
        sz = _HALF * 16
        pltpu.sync_copy(oi_v, omap_h.at[pl.ds(off, sz)])
        pltpu.sync_copy(ox_v, ox_h.at[pl.ds(off, sz)])
        pltpu.sync_copy(oy_v, oy_h.at[pl.ds(off, sz)])
        pltpu.sync_copy(oz_v, oz_h.at[pl.ds(off, sz)])


_mesh = plsc.VectorSubcoreMesh(core_axis_name="c", subcore_axis_name="s")

_sc_call = pl.kernel(
    _sc_body,
    out_type=[
        jax.ShapeDtypeStruct((_NQ * 16,), jnp.int32),
        jax.ShapeDtypeStruct((_NQ * 16,), jnp.float32),
        jax.ShapeDtypeStruct((_NQ * 16,), jnp.float32),
        jax.ShapeDtypeStruct((_NQ * 16,), jnp.float32),
    ],
    mesh=_mesh,
    compiler_params=pltpu.CompilerParams(use_tc_tiling_on_sc=False, needs_layout_passes=False),
    scratch_types=[
        pltpu.VMEM((_NP,), jnp.float32),
        pltpu.VMEM((_NP,), jnp.float32),
        pltpu.VMEM((_NP,), jnp.float32),
        pltpu.VMEM((_NP,), jnp.float32),
        pltpu.VMEM((_NP,), jnp.int32),
        pltpu.VMEM((_NCELL + 24,), jnp.int32),
        pltpu.VMEM((_QPW + 16,), jnp.float32),
        pltpu.VMEM((_QPW + 16,), jnp.float32),
        pltpu.VMEM((_QPW + 16,), jnp.float32),
        pltpu.VMEM((_NCOLP + 16,), jnp.int32),
        pltpu.VMEM((_NCOLP + 16,), jnp.int32),
        pltpu.VMEM((_NCOLP + 16,), jnp.int32),
        pltpu.VMEM((_NCOLP + 16,), jnp.float32),
        pltpu.VMEM((40,), jnp.int32),
        pltpu.VMEM((40,), jnp.int32),
        pltpu.VMEM((_HALF * 16,), jnp.int32),
        pltpu.VMEM((_HALF * 16,), jnp.float32),
        pltpu.VMEM((_HALF * 16,), jnp.float32),
        pltpu.VMEM((_HALF * 16,), jnp.float32),
    ],
)


@jax.jit
def kernel(x, p_grid):
    pts = x[0]
    ci = jnp.clip(jnp.floor(pts * np.float32(_C)).astype(jnp.int32),
                  0, _C - 1)
    cid = (ci[:, 0] * _C + ci[:, 1]) * _C + ci[:, 2]
    order = jnp.argsort(cid).astype(jnp.int32)
    sp = jnp.take(pts, order, axis=0)
    cid_s = jnp.take(cid, order)
    starts = jnp.searchsorted(
        cid_s, jnp.arange(_NCELL + 1, dtype=jnp.int32)).astype(jnp.int32)
    starts = jnp.concatenate(
        [starts, jnp.full((7,), _NP, jnp.int32)])
    q = p_grid.reshape(-1, 3)

    omap, ox, oy, oz = _sc_call(
        jnp.copy(sp[:, 0]), jnp.copy(sp[:, 1]),
        jnp.copy(sp[:, 2]), order, starts,
        jnp.copy(q[:, 0]), jnp.copy(q[:, 1]),
        jnp.copy(q[:, 2]),
        jnp.asarray(_CDX), jnp.asarray(_CDY), jnp.asarray(_CM2),
        jnp.asarray(_CLB2), jnp.asarray(_ZLUT), jnp.asarray(_CCNT))

    mapping = omap.reshape(_NQ, 16)[:, :_K][None]
    outputs = jnp.stack(
        [ox.reshape(_NQ, 16)[:, :_K], oy.reshape(_NQ, 16)[:, :_K],
         oz.reshape(_NQ, 16)[:, :_K]], axis=-1)[None]
    return (mapping, outputs)


# speedup vs baseline: 2.3581x; 1.0205x over previous
import functools

import jax
import jax.numpy as jnp
import numpy as np
from jax import lax
from jax.experimental import pallas as pl
from jax.experimental.pallas import tpu as pltpu
from jax.experimental.pallas import tpu_sc as plsc

_C = 16
_NCELL = _C * _C * _C
_NP = 16384
_NQ = 32768
_K = 10
_R2 = np.float32(0.0625)
_INF = np.float32(np.inf)
_CELL2 = np.float32(1.0 / (_C * _C * _C * _C))
_NW = 32
_QPW = _NQ // _NW
_HALF = _QPW // 2

_cols = []
for _dx in range(-5, 6):
    for _dy in range(-5, 6):
        _m1 = max(abs(_dx) - 1, 0)
        _m2 = max(abs(_dy) - 1, 0)
        _mm = _m1 * _m1 + _m2 * _m2
        if _mm <= 22:
            _cols.append((_mm, _dx, _dy))
_cols.sort()
_NCOL = len(_cols)
_NCOLP = ((_NCOL + 7) // 8) * 8
_CDX = np.array([c[1] for c in _cols] + [0] * (_NCOLP - _NCOL), np.int32)
_CDY = np.array([c[2] for c in _cols] + [0] * (_NCOLP - _NCOL), np.int32)
_CM2 = np.array([c[0] for c in _cols] + [0] * (_NCOLP - _NCOL), np.int32)
_CLB2 = np.array(
    [c[0] * float(_CELL2) for c in _cols] + [np.inf] * (_NCOLP - _NCOL),
    np.float32)
_ZLUT = np.array([int(np.floor(np.sqrt(r))) for r in range(24)], np.int32)
_CCNT = np.array([sum(1 for c in _cols if c[0] <= t) for t in range(24)],
                 np.int32)

_IOTA = None


def _sload(ref, i):
    return ref[pl.ds(i, 16)][0]


def _rne_bf16(v):
    b = lax.bitcast_convert_type(v, jnp.uint32)
    r = (b + jnp.uint32(0x7FFF) + ((b >> jnp.uint32(16)) & jnp.uint32(1)))
    r = r & jnp.uint32(0xFFFF0000)
    return lax.bitcast_convert_type(r, jnp.float32)


def _sc_body(spx_h, spy_h, spz_h, pidx_h, starts_h, qx_h, qy_h, qz_h,
             cdx_h, cdy_h, cm2_h, clb2_h, lut_h, ccnt_h,
             omap_h, ox_h, oy_h, oz_h,
             px_v, py_v, pz_v, pn_v, pidx_v, starts_v,
             qx_v, qy_v, qz_v,
             cdx_v, cdy_v, cm2_v, clb2_v, lut_v, ccnt_v,
             oi_v, ox_v, oy_v, oz_v):
    wid = lax.axis_index("s") * 2 + lax.axis_index("c")
    qbase = wid * _QPW

    pltpu.sync_copy(spx_h, px_v)
    pltpu.sync_copy(spy_h, py_v)
    pltpu.sync_copy(spz_h, pz_v)
    pltpu.sync_copy(pidx_h, pidx_v)
    pltpu.sync_copy(starts_h, starts_v.at[pl.ds(0, _NCELL + 8)])
    pltpu.sync_copy(qx_h.at[pl.ds(qbase, _QPW)], qx_v.at[pl.ds(0, _QPW)])
    pltpu.sync_copy(qy_h.at[pl.ds(qbase, _QPW)], qy_v.at[pl.ds(0, _QPW)])
    pltpu.sync_copy(qz_h.at[pl.ds(qbase, _QPW)], qz_v.at[pl.ds(0, _QPW)])
    pltpu.sync_copy(cdx_h, cdx_v.at[pl.ds(0, _NCOLP)])
    pltpu.sync_copy(cdy_h, cdy_v.at[pl.ds(0, _NCOLP)])
    pltpu.sync_copy(cm2_h, cm2_v.at[pl.ds(0, _NCOLP)])
    pltpu.sync_copy(clb2_h, clb2_v.at[pl.ds(0, _NCOLP)])
    pltpu.sync_copy(lut_h, lut_v.at[pl.ds(0, 24)])
    pltpu.sync_copy(ccnt_h, ccnt_v.at[pl.ds(0, 24)])

    iota = lax.iota(jnp.int32, 16)
    rank_mask = iota < _K

    def _pn_step(i, smax):
        sl = pl.ds(i * 16, 16)
        px = px_v[sl]
        py = py_v[sl]
        pz = pz_v[sl]
        pn_v[sl] = (px * px + pz * pz) + py * py
        sp = (jnp.abs(px - _rne_bf16(px)) + jnp.abs(py - _rne_bf16(py))
              + jnp.abs(pz - _rne_bf16(pz)))
        return jnp.maximum(smax, sp)

    smax_vec = lax.fori_loop(0, _NP // 16, _pn_step,
                             jnp.zeros((16,), jnp.float32))
    for _sh in (8, 4, 2, 1):
        smax_vec = jnp.maximum(smax_vec, jnp.take(smax_vec, iota ^ _sh))
    e_base = 2.0 * smax_vec[0] + np.float32(1e-6)

    def _merge(keys, vals, nk, nv):
        sk, sv = plsc.sort_key_val(nk, nv)
        rk = lax.rev(sk, (0,))
        rv = lax.rev(sv, (0,))
        take = keys <= rk
        mk = jnp.where(take, keys, rk)
        mv = jnp.where(take, vals, rv)
        out = plsc.sort_key_val(mk, mv)
        return out[0], out[1]

    def _w10(keys):
        return keys[9]

    def _do_query(qi, ql):
        qx = _sload(qx_v, qi)
        qy = _sload(qy_v, qi)
        qz = _sload(qz_v, qi)
        qxv = jnp.full((16,), qx)
        qyv = jnp.full((16,), qy)
        qzv = jnp.full((16,), qz)
        bqx = _rne_bf16(qxv)
        bqy = _rne_bf16(qyv)
        bqz = _rne_bf16(qzv)
        qnv = (qxv * qxv + qzv * qzv) + qyv * qyv
        sqv = (jnp.abs(qxv - bqx) + jnp.abs(qyv - bqy)
               + jnp.abs(qzv - bqz))
        e_q = e_base + 2.0 * sqv[0]
        cx = jnp.clip((qx * np.float32(_C)).astype(jnp.int32), 0, _C - 1)
        cy = jnp.clip((qy * np.float32(_C)).astype(jnp.int32), 0, _C - 1)
        cz = jnp.clip((qz * np.float32(_C)).astype(jnp.int32), 0, _C - 1)

        def _scan_range(s, e, keys, vals):
            ntrip = (e - s + 15) >> 4

            def _inner_body(it, st):
                keys, vals = st
                j = s + it * 16
                lanes = j + iota
                inb = lanes < e
                lc = jnp.minimum(lanes, e - 1)
                px = plsc.load_gather(px_v, [lc])
                py = plsc.load_gather(py_v, [lc])
                pz = plsc.load_gather(pz_v, [lc])
                pn = plsc.load_gather(pn_v, [lc])
                p0 = bqx * _rne_bf16(px)
                p1 = bqy * _rne_bf16(py)
                p2 = bqz * _rne_bf16(pz)
                s1 = p0 + p1
                bb = s1 - p0
                er1 = (p0 - (s1 - bb)) + (p1 - bb)
                s2 = s1 + p2
                bb2 = s2 - s1
                er2 = (s1 - (s2 - bb2)) + (p2 - bb2)
                mm = s2 + (er1 + er2)
                d2 = (qnv + pn) - 2.0 * mm
                key = jnp.where(inb & (d2 <= _R2), d2, _INF)
                beats = plsc.all_reduce_population_count(
                    (key <= jnp.full((16,), _w10(keys))) & (key < _INF))

                def _mb(_, st2):
                    return _merge(st2[0], st2[1], key, lc)

                keys, vals = lax.fori_loop(
                    0, jnp.minimum(beats[0], 1), _mb, (keys, vals))
                return keys, vals

            return lax.fori_loop(0, ntrip, _inner_body, (keys, vals))

        colc0 = (cx * _C + cy) * _C
        zs0 = jnp.maximum(cz - 2, 0)
        zs1 = jnp.minimum(cz + 2, _C - 1)
        zsel0 = colc0 + jnp.where(iota < 1, zs0, zs1 + 1)
        sev0 = plsc.load_gather(starts_v, [zsel0])
        seedk, _sv = _scan_range(sev0[0], sev0[1],
                                 jnp.full((16,), _INF),
                                 jnp.zeros((16,), jnp.int32))
        w10_cap = seedk[9]

        def _grp_body(g, st):
            keys, vals = st
            base = g * 16
            dxv = cdx_v[pl.ds(base, 16)]
            dyv = cdy_v[pl.ds(base, 16)]
            m2v = cm2_v[pl.ds(base, 16)]
            lbv = clb2_v[pl.ds(base, 16)]
            ixv = cx + dxv
            iyv = cy + dyv
            okv = (ixv >= 0) & (ixv < _C) & (iyv >= 0) & (iyv < _C)
            thr = jnp.minimum(jnp.minimum(_w10(keys), w10_cap), _R2) + e_q
            actv = okv & (lbv <= thr)
            tcs = jnp.clip((thr * np.float32(256.0)).astype(jnp.int32) + 1,
                           0, 23)
            remv = jnp.clip(tcs - m2v, 0, 23)
            rzv = plsc.load_gather(lut_v, [remv]) + 1
            z0v = jnp.maximum(cz - rzv, 0)
            z1v = jnp.minimum(cz + rzv, _C - 1)
            cbv = (jnp.clip(ixv, 0, _C - 1) * _C
                   + jnp.clip(iyv, 0, _C - 1)) * _C
            sv = plsc.load_gather(starts_v, [cbv + z0v])
            ev = plsc.load_gather(starts_v, [cbv + z1v + 1])
            ev = jnp.where(actv, ev, sv)
            lenv = ev - sv
            pref = plsc.cumsum(lenv)
            m_tot = pref[15]
            nb = (m_tot + 15) >> 4

            def _t_body(t, st2):
                keys, vals = st2
                flat = t * 16 + iota
                c = jnp.zeros((16,), jnp.int32)
                for _step in (8, 4, 2, 1):
                    pv = jnp.take(pref, c + (_step - 1))
                    c = c + jnp.where(pv <= flat, _step, 0)
                cc = jnp.minimum(c, 15)
                excl = jnp.where(c == 0, 0,
                                 jnp.take(pref, jnp.maximum(c - 1, 0)))
                inb = flat < jnp.full((16,), m_tot)
                lc = jnp.minimum(jnp.take(sv, cc) + (flat - excl), _NP - 1)
                px = plsc.load_gather(px_v, [lc])
                py = plsc.load_gather(py_v, [lc])
                pz = plsc.load_gather(pz_v, [lc])
                pn = plsc.load_gather(pn_v, [lc])
                p0 = bqx * _rne_bf16(px)
                p1 = bqy * _rne_bf16(py)
                p2 = bqz * _rne_bf16(pz)
                s1 = p0 + p1
                bb = s1 - p0
                er1 = (p0 - (s1 - bb)) + (p1 - bb)
                s2 = s1 + p2
                bb2 = s2 - s1
                er2 = (s1 - (s2 - bb2)) + (p2 - bb2)
                mm = s2 + (er1 + er2)
                d2 = (qnv + pn) - 2.0 * mm
                key = jnp.where(inb & (d2 <= _R2), d2, _INF)
                beats = plsc.all_reduce_population_count(
                    (key <= jnp.full((16,), _w10(keys))) & (key < _INF))

                def _mb(_, st3):
                    return _merge(st3[0], st3[1], key, lc)

                keys, vals = lax.fori_loop(
                    0, jnp.minimum(beats[0], 1), _mb, (keys, vals))
                return keys, vals

            return lax.fori_loop(0, nb, _t_body, (keys, vals))

        keys0 = jnp.full((16,), _INF)
        vals0 = jnp.zeros((16,), jnp.int32)
        keys, vals = _grp_body(0, (keys0, vals0))
        thr0 = jnp.minimum(jnp.minimum(_w10(keys), w10_cap), _R2) + e_q
        tc0 = jnp.clip((thr0 * np.float32(256.0)).astype(jnp.int32) + 1,
                       0, 23)
        n_act = _sload(ccnt_v, tc0)
        n_grp = (n_act + 15) >> 4
        keys, vals = lax.fori_loop(1, jnp.minimum(n_grp, 2), _grp_body,
                                   (keys, vals))
        thr1 = jnp.minimum(jnp.minimum(_w10(keys), w10_cap), _R2) + e_q
        tc1 = jnp.clip((thr1 * np.float32(256.0)).astype(jnp.int32) + 1,
                       0, 23)
        n_grp2 = (_sload(ccnt_v, tc1) + 15) >> 4
        keys, vals = lax.fori_loop(2, n_grp2, _grp_body, (keys, vals))

        shifted = jnp.take(keys, jnp.minimum(iota + 1, 15))
        tiec = plsc.all_reduce_population_count(
            (keys == shifted) & (iota < 15) & (shifted < _INF))

        def _fix(_, vv):
            oidx0 = plsc.load_gather(pidx_v, [vv])
            r = jnp.zeros((16,), jnp.int32)
            for k in range(16):
                kv = jnp.take(keys, jnp.full((16,), k, jnp.int32))
                r = r + (kv < keys).astype(jnp.int32)
            surrogate = (r << 14) | oidx0
            sout = plsc.sort_key_val(surrogate, vv)
            return sout[1]

        vals = lax.fori_loop(0, jnp.minimum(tiec[0], 1), _fix, vals)

        oidx = plsc.load_gather(pidx_v, [vals])
        pxo = plsc.load_gather(px_v, [vals])
        pyo = plsc.load_gather(py_v, [vals])
        pzo = plsc.load_gather(pz_v, [vals])
        valid = (keys <= _R2) & rank_mask
        sl = pl.ds(ql * 16, 16)
        oi_v[sl] = jnp.where(valid, oidx, 0)
        ox_v[sl] = jnp.where(valid, pxo, np.float32(0.0))
        oy_v[sl] = jnp.where(valid, pyo, np.float32(0.0))
        oz_v[sl] = jnp.where(valid, pzo, np.float32(0.0))

    for half in range(2):
        def _qstep(ql, _c, half=half):
            _do_query(half * _HALF + ql, ql)
            return _c

        lax.fori_loop(0, _HALF, _qstep, 0)
        off = (qbase + half * _HALF) * 16
        sz = _HALF * 16
        pltpu.sync_copy(oi_v, omap_h.at[pl.ds(off, sz)])
        pltpu.sync_copy(ox_v, ox_h.at[pl.ds(off, sz)])
        pltpu.sync_copy(oy_v, oy_h.at[pl.ds(off, sz)])
        pltpu.sync_copy(oz_v, oz_h.at[pl.ds(off, sz)])


_mesh = plsc.VectorSubcoreMesh(core_axis_name="c", subcore_axis_name="s")

_sc_call = pl.kernel(
    _sc_body,
    out_type=[
        jax.ShapeDtypeStruct((_NQ * 16,), jnp.int32),
        jax.ShapeDtypeStruct((_NQ * 16,), jnp.float32),
        jax.ShapeDtypeStruct((_NQ * 16,), jnp.float32),
        jax.ShapeDtypeStruct((_NQ * 16,), jnp.float32),
    ],
    mesh=_mesh,
    compiler_params=pltpu.CompilerParams(use_tc_tiling_on_sc=False, needs_layout_passes=False),
    scratch_types=[
        pltpu.VMEM((_NP,), jnp.float32),
        pltpu.VMEM((_NP,), jnp.float32),
        pltpu.VMEM((_NP,), jnp.float32),
        pltpu.VMEM((_NP,), jnp.float32),
        pltpu.VMEM((_NP,), jnp.int32),
        pltpu.VMEM((_NCELL + 24,), jnp.int32),
        pltpu.VMEM((_QPW + 16,), jnp.float32),
        pltpu.VMEM((_QPW + 16,), jnp.float32),
        pltpu.VMEM((_QPW + 16,), jnp.float32),
        pltpu.VMEM((_NCOLP + 16,), jnp.int32),
        pltpu.VMEM((_NCOLP + 16,), jnp.int32),
        pltpu.VMEM((_NCOLP + 16,), jnp.int32),
        pltpu.VMEM((_NCOLP + 16,), jnp.float32),
        pltpu.VMEM((40,), jnp.int32),
        pltpu.VMEM((40,), jnp.int32),
        pltpu.VMEM((_HALF * 16,), jnp.int32),
        pltpu.VMEM((_HALF * 16,), jnp.float32),
        pltpu.VMEM((_HALF * 16,), jnp.float32),
        pltpu.VMEM((_HALF * 16,), jnp.float32),
    ],
)


@jax.jit
def kernel(x, p_grid):
    pts = x[0]
    ci = jnp.clip(jnp.floor(pts * np.float32(_C)).astype(jnp.int32),
                  0, _C - 1)
    cid = (ci[:, 0] * _C + ci[:, 1]) * _C + ci[:, 2]
    order = jnp.argsort(cid).astype(jnp.int32)
    sp = jnp.take(pts, order, axis=0)
    cid_s = jnp.take(cid, order)
    starts = jnp.searchsorted(
        cid_s, jnp.arange(_NCELL + 1, dtype=jnp.int32)).astype(jnp.int32)
    starts = jnp.concatenate(
        [starts, jnp.full((7,), _NP, jnp.int32)])
    q = p_grid.reshape(-1, 3)

    omap, ox, oy, oz = _sc_call(
        jnp.copy(sp[:, 0]), jnp.copy(sp[:, 1]),
        jnp.copy(sp[:, 2]), order, starts,
        jnp.copy(q[:, 0]), jnp.copy(q[:, 1]),
        jnp.copy(q[:, 2]),
        jnp.asarray(_CDX), jnp.asarray(_CDY), jnp.asarray(_CM2),
        jnp.asarray(_CLB2), jnp.asarray(_ZLUT), jnp.asarray(_CCNT))

    mapping = omap.reshape(_NQ, 16)[:, :_K][None]
    outputs = jnp.stack(
        [ox.reshape(_NQ, 16)[:, :_K], oy.reshape(_NQ, 16)[:, :_K],
         oz.reshape(_NQ, 16)[:, :_K]], axis=-1)[None]
    return (mapping, outputs)


# speedup vs baseline: 2.3582x; 1.0001x over previous
import jax
import jax.numpy as jnp
import numpy as np
from jax import lax
from jax.experimental import pallas as pl
from jax.experimental.pallas import tpu as pltpu
from jax.experimental.pallas import tpu_sc as plsc

_C = 16
_NCELL = _C * _C * _C
_NP = 16384
_NQ = 32768
_K = 10
_R2 = np.float32(0.0625)
_INF = np.float32(np.inf)
_CELL2 = np.float32(1.0 / (_C * _C * _C * _C))
_NW = 32
_QPW = _NQ // _NW
_HALF = _QPW // 2

_cols = []
for _dx in range(-5, 6):
    for _dy in range(-5, 6):
        _m1 = max(abs(_dx) - 1, 0)
        _m2 = max(abs(_dy) - 1, 0)
        _mm = _m1 * _m1 + _m2 * _m2
        if _mm <= 22:
            _cols.append((_mm, _dx, _dy))
_cols.sort()
_NCOL = len(_cols)
_NCOLP = ((_NCOL + 7) // 8) * 8
_CDX = np.array([c[1] for c in _cols] + [0] * (_NCOLP - _NCOL), np.int32)
_CDY = np.array([c[2] for c in _cols] + [0] * (_NCOLP - _NCOL), np.int32)
_CM2 = np.array([c[0] for c in _cols] + [0] * (_NCOLP - _NCOL), np.int32)
_CLB2 = np.array(
    [c[0] * float(_CELL2) for c in _cols] + [np.inf] * (_NCOLP - _NCOL),
    np.float32)
_ZLUT = np.array([int(np.floor(np.sqrt(r))) for r in range(24)], np.int32)
_CCNT = np.array([sum(1 for c in _cols if c[0] <= t) for t in range(24)],
                 np.int32)

def _sload(ref, i):
    return ref[pl.ds(i, 16)][0]


def _rne_bf16(v):
    b = lax.bitcast_convert_type(v, jnp.uint32)
    r = (b + jnp.uint32(0x7FFF) + ((b >> jnp.uint32(16)) & jnp.uint32(1)))
    r = r & jnp.uint32(0xFFFF0000)
    return lax.bitcast_convert_type(r, jnp.float32)


def _sc_body(spx_h, spy_h, spz_h, pidx_h, starts_h, qx_h, qy_h, qz_h,
             cdx_h, cdy_h, cm2_h, clb2_h, lut_h, ccnt_h,
             omap_h, ox_h, oy_h, oz_h,
             px_v, py_v, pz_v, pn_v, pidx_v, starts_v,
             qx_v, qy_v, qz_v,
             cdx_v, cdy_v, cm2_v, clb2_v, lut_v, ccnt_v,
             oi_v, ox_v, oy_v, oz_v):
    wid = lax.axis_index("s") * 2 + lax.axis_index("c")
    qbase = wid * _QPW

    pltpu.sync_copy(spx_h, px_v)
    pltpu.sync_copy(spy_h, py_v)
    pltpu.sync_copy(spz_h, pz_v)
    pltpu.sync_copy(pidx_h, pidx_v)
    pltpu.sync_copy(starts_h, starts_v.at[pl.ds(0, _NCELL + 8)])
    pltpu.sync_copy(qx_h.at[pl.ds(qbase, _QPW)], qx_v.at[pl.ds(0, _QPW)])
    pltpu.sync_copy(qy_h.at[pl.ds(qbase, _QPW)], qy_v.at[pl.ds(0, _QPW)])
    pltpu.sync_copy(qz_h.at[pl.ds(qbase, _QPW)], qz_v.at[pl.ds(0, _QPW)])
    pltpu.sync_copy(cdx_h, cdx_v.at[pl.ds(0, _NCOLP)])
    pltpu.sync_copy(cdy_h, cdy_v.at[pl.ds(0, _NCOLP)])
    pltpu.sync_copy(cm2_h, cm2_v.at[pl.ds(0, _NCOLP)])
    pltpu.sync_copy(clb2_h, clb2_v.at[pl.ds(0, _NCOLP)])
    pltpu.sync_copy(lut_h, lut_v.at[pl.ds(0, 24)])
    pltpu.sync_copy(ccnt_h, ccnt_v.at[pl.ds(0, 24)])

    iota = lax.iota(jnp.int32, 16)
    rank_mask = iota < _K

    def _pn_step(i, smax):
        sl = pl.ds(i * 16, 16)
        px = px_v[sl]
        py = py_v[sl]
        pz = pz_v[sl]
        pn_v[sl] = (px * px + pz * pz) + py * py
        sp = (jnp.abs(px - _rne_bf16(px)) + jnp.abs(py - _rne_bf16(py))
              + jnp.abs(pz - _rne_bf16(pz)))
        return jnp.maximum(smax, sp)

    smax_vec = lax.fori_loop(0, _NP // 16, _pn_step,
                             jnp.zeros((16,), jnp.float32))
    for _sh in (8, 4, 2, 1):
        smax_vec = jnp.maximum(smax_vec, jnp.take(smax_vec, iota ^ _sh))
    e_base = 2.0 * smax_vec[0] + np.float32(1e-6)

    def _merge(keys, vals, nk, nv):
        sk, sv = plsc.sort_key_val(nk, nv)
        rk = lax.rev(sk, (0,))
        rv = lax.rev(sv, (0,))
        take = keys <= rk
        mk = jnp.where(take, keys, rk)
        mv = jnp.where(take, vals, rv)
        out = plsc.sort_key_val(mk, mv)
        return out[0], out[1]

    def _w10(keys):
        return keys[9]

    def _do_query(qi, ql):
        qx = _sload(qx_v, qi)
        qy = _sload(qy_v, qi)
        qz = _sload(qz_v, qi)
        qxv = jnp.full((16,), qx)
        qyv = jnp.full((16,), qy)
        qzv = jnp.full((16,), qz)
        bqx = _rne_bf16(qxv)
        bqy = _rne_bf16(qyv)
        bqz = _rne_bf16(qzv)
        qnv = (qxv * qxv + qzv * qzv) + qyv * qyv
        sqv = (jnp.abs(qxv - bqx) + jnp.abs(qyv - bqy)
               + jnp.abs(qzv - bqz))
        e_q = e_base + 2.0 * sqv[0]
        cx = jnp.clip((qx * np.float32(_C)).astype(jnp.int32), 0, _C - 1)
        cy = jnp.clip((qy * np.float32(_C)).astype(jnp.int32), 0, _C - 1)
        cz = jnp.clip((qz * np.float32(_C)).astype(jnp.int32), 0, _C - 1)

        def _scan_range(s, e, keys, vals):
            ntrip = (e - s + 15) >> 4

            def _inner_body(it, st):
                keys, vals = st
                j = s + it * 16
                lanes = j + iota
                inb = lanes < e
                lc = jnp.minimum(lanes, e - 1)
                px = plsc.load_gather(px_v, [lc])
                py = plsc.load_gather(py_v, [lc])
                pz = plsc.load_gather(pz_v, [lc])
                pn = plsc.load_gather(pn_v, [lc])
                p0 = bqx * _rne_bf16(px)
                p1 = bqy * _rne_bf16(py)
                p2 = bqz * _rne_bf16(pz)
                s1 = p0 + p1
                bb = s1 - p0
                er1 = (p0 - (s1 - bb)) + (p1 - bb)
                s2 = s1 + p2
                bb2 = s2 - s1
                er2 = (s1 - (s2 - bb2)) + (p2 - bb2)
                mm = s2 + (er1 + er2)
                d2 = (qnv + pn) - 2.0 * mm
                key = jnp.where(inb & (d2 <= _R2), d2, _INF)
                beats = plsc.all_reduce_population_count(
                    (key <= jnp.full((16,), _w10(keys))) & (key < _INF))

                def _mb(_, st2):
                    return _merge(st2[0], st2[1], key, lc)

                keys, vals = lax.fori_loop(
                    0, jnp.minimum(beats[0], 1), _mb, (keys, vals))
                return keys, vals

            return lax.fori_loop(0, ntrip, _inner_body, (keys, vals))

        colc0 = (cx * _C + cy) * _C
        zs0 = jnp.maximum(cz - 2, 0)
        zs1 = jnp.minimum(cz + 2, _C - 1)
        zsel0 = colc0 + jnp.where(iota < 1, zs0, zs1 + 1)
        sev0 = plsc.load_gather(starts_v, [zsel0])
        seedk, _sv = _scan_range(sev0[0], sev0[1],
                                 jnp.full((16,), _INF),
                                 jnp.zeros((16,), jnp.int32))
        w10_cap = seedk[9]

        def _grp_body(g, st):
            keys, vals = st
            base = g * 16
            dxv = cdx_v[pl.ds(base, 16)]
            dyv = cdy_v[pl.ds(base, 16)]
            m2v = cm2_v[pl.ds(base, 16)]
            lbv = clb2_v[pl.ds(base, 16)]
            ixv = cx + dxv
            iyv = cy + dyv
            okv = (ixv >= 0) & (ixv < _C) & (iyv >= 0) & (iyv < _C)
            thr = jnp.minimum(jnp.minimum(_w10(keys), w10_cap), _R2) + e_q
            actv = okv & (lbv <= thr)
            tcs = jnp.clip((thr * np.float32(256.0)).astype(jnp.int32) + 1,
                           0, 23)
            remv = jnp.clip(tcs - m2v, 0, 23)
            rzv = plsc.load_gather(lut_v, [remv]) + 1
            z0v = jnp.maximum(cz - rzv, 0)
            z1v = jnp.minimum(cz + rzv, _C - 1)
            cbv = (jnp.clip(ixv, 0, _C - 1) * _C
                   + jnp.clip(iyv, 0, _C - 1)) * _C
            sv = plsc.load_gather(starts_v, [cbv + z0v])
            ev = plsc.load_gather(starts_v, [cbv + z1v + 1])
            ev = jnp.where(actv, ev, sv)
            lenv = ev - sv
            pref = plsc.cumsum(lenv)
            m_tot = pref[15]
            nb = (m_tot + 15) >> 4

            def _t_body(t, st2):
                keys, vals = st2
                flat = t * 16 + iota
                c = jnp.zeros((16,), jnp.int32)
                for _step in (8, 4, 2, 1):
                    pv = jnp.take(pref, c + (_step - 1))
                    c = c + jnp.where(pv <= flat, _step, 0)
                cc = jnp.minimum(c, 15)
                excl = jnp.where(c == 0, 0,
                                 jnp.take(pref, jnp.maximum(c - 1, 0)))
                inb = flat < jnp.full((16,), m_tot)
                lc = jnp.minimum(jnp.take(sv, cc) + (flat - excl), _NP - 1)
                px = plsc.load_gather(px_v, [lc])
                py = plsc.load_gather(py_v, [lc])
                pz = plsc.load_gather(pz_v, [lc])
                pn = plsc.load_gather(pn_v, [lc])
                p0 = bqx * _rne_bf16(px)
                p1 = bqy * _rne_bf16(py)
                p2 = bqz * _rne_bf16(pz)
                s1 = p0 + p1
                bb = s1 - p0
                er1 = (p0 - (s1 - bb)) + (p1 - bb)
                s2 = s1 + p2
                bb2 = s2 - s1
                er2 = (s1 - (s2 - bb2)) + (p2 - bb2)
                mm = s2 + (er1 + er2)
                d2 = (qnv + pn) - 2.0 * mm
                key = jnp.where(inb & (d2 <= _R2), d2, _INF)
                beats = plsc.all_reduce_population_count(
                    (key <= jnp.full((16,), _w10(keys))) & (key < _INF))

                def _mb(_, st3):
                    return _merge(st3[0], st3[1], key, lc)

                keys, vals = lax.fori_loop(
                    0, jnp.minimum(beats[0], 1), _mb, (keys, vals))
                return keys, vals

            return lax.fori_loop(0, nb, _t_body, (keys, vals))

        keys0 = jnp.full((16,), _INF)
        vals0 = jnp.zeros((16,), jnp.int32)
        keys, vals = _grp_body(0, (keys0, vals0))
        thr0 = jnp.minimum(jnp.minimum(_w10(keys), w10_cap), _R2) + e_q
        tc0 = jnp.clip((thr0 * np.float32(256.0)).astype(jnp.int32) + 1,
                       0, 23)
        n_act = _sload(ccnt_v, tc0)
        n_grp = (n_act + 15) >> 4
        keys, vals = lax.fori_loop(1, jnp.minimum(n_grp, 2), _grp_body,
                                   (keys, vals))
        thr1 = jnp.minimum(jnp.minimum(_w10(keys), w10_cap), _R2) + e_q
        tc1 = jnp.clip((thr1 * np.float32(256.0)).astype(jnp.int32) + 1,
                       0, 23)
        n_grp2 = (_sload(ccnt_v, tc1) + 15) >> 4
        keys, vals = lax.fori_loop(2, n_grp2, _grp_body, (keys, vals))

        shifted = jnp.take(keys, jnp.minimum(iota + 1, 15))
        tiec = plsc.all_reduce_population_count(
            (keys == shifted) & (iota < 15) & (shifted < _INF))

        def _fix(_, vv):
            oidx0 = plsc.load_gather(pidx_v, [vv])
            r = jnp.zeros((16,), jnp.int32)
            for k in range(16):
                kv = jnp.take(keys, jnp.full((16,), k, jnp.int32))
                r = r + (kv < keys).astype(jnp.int32)
            surrogate = (r << 14) | oidx0
            sout = plsc.sort_key_val(surrogate, vv)
            return sout[1]

        vals = lax.fori_loop(0, jnp.minimum(tiec[0], 1), _fix, vals)

        oidx = plsc.load_gather(pidx_v, [vals])
        pxo = plsc.load_gather(px_v, [vals])
        pyo = plsc.load_gather(py_v, [vals])
        pzo = plsc.load_gather(pz_v, [vals])
        valid = (keys <= _R2) & rank_mask
        sl = pl.ds(ql * 16, 16)
        oi_v[sl] = jnp.where(valid, oidx, 0)
        ox_v[sl] = jnp.where(valid, pxo, np.float32(0.0))
        oy_v[sl] = jnp.where(valid, pyo, np.float32(0.0))
        oz_v[sl] = jnp.where(valid, pzo, np.float32(0.0))

    for half in range(2):
        def _qstep(ql, _c, half=half):
            _do_query(half * _HALF + ql, ql)
            return _c

        lax.fori_loop(0, _HALF, _qstep, 0)
        off = (qbase + half * _HALF) * 16
        sz = _HALF * 16
        pltpu.sync_copy(oi_v, omap_h.at[pl.ds(off, sz)])
        pltpu.sync_copy(ox_v, ox_h.at[pl.ds(off, sz)])
        pltpu.sync_copy(oy_v, oy_h.at[pl.ds(off, sz)])
        pltpu.sync_copy(oz_v, oz_h.at[pl.ds(off, sz)])


_mesh = plsc.VectorSubcoreMesh(core_axis_name="c", subcore_axis_name="s")

_sc_call = pl.kernel(
    _sc_body,
    out_type=[
        jax.ShapeDtypeStruct((_NQ * 16,), jnp.int32),
        jax.ShapeDtypeStruct((_NQ * 16,), jnp.float32),
        jax.ShapeDtypeStruct((_NQ * 16,), jnp.float32),
        jax.ShapeDtypeStruct((_NQ * 16,), jnp.float32),
    ],
    mesh=_mesh,
    compiler_params=pltpu.CompilerParams(use_tc_tiling_on_sc=False, needs_layout_passes=False),
    scratch_types=[
        pltpu.VMEM((_NP,), jnp.float32),
        pltpu.VMEM((_NP,), jnp.float32),
        pltpu.VMEM((_NP,), jnp.float32),
        pltpu.VMEM((_NP,), jnp.float32),
        pltpu.VMEM((_NP,), jnp.int32),
        pltpu.VMEM((_NCELL + 24,), jnp.int32),
        pltpu.VMEM((_QPW + 16,), jnp.float32),
        pltpu.VMEM((_QPW + 16,), jnp.float32),
        pltpu.VMEM((_QPW + 16,), jnp.float32),
        pltpu.VMEM((_NCOLP + 16,), jnp.int32),
        pltpu.VMEM((_NCOLP + 16,), jnp.int32),
        pltpu.VMEM((_NCOLP + 16,), jnp.int32),
        pltpu.VMEM((_NCOLP + 16,), jnp.float32),
        pltpu.VMEM((40,), jnp.int32),
        pltpu.VMEM((40,), jnp.int32),
        pltpu.VMEM((_HALF * 16,), jnp.int32),
        pltpu.VMEM((_HALF * 16,), jnp.float32),
        pltpu.VMEM((_HALF * 16,), jnp.float32),
        pltpu.VMEM((_HALF * 16,), jnp.float32),
    ],
)


@jax.jit
def kernel(x, p_grid):
    pts = x[0]
    ci = jnp.clip(jnp.floor(pts * np.float32(_C)).astype(jnp.int32),
                  0, _C - 1)
    cid = (ci[:, 0] * _C + ci[:, 1]) * _C + ci[:, 2]
    order = jnp.argsort(cid).astype(jnp.int32)
    sp = jnp.take(pts, order, axis=0)
    cid_s = jnp.take(cid, order)
    starts = jnp.searchsorted(
        cid_s, jnp.arange(_NCELL + 1, dtype=jnp.int32)).astype(jnp.int32)
    starts = jnp.concatenate(
        [starts, jnp.full((7,), _NP, jnp.int32)])
    q = p_grid.reshape(-1, 3)

    omap, ox, oy, oz = _sc_call(
        jnp.copy(sp[:, 0]), jnp.copy(sp[:, 1]),
        jnp.copy(sp[:, 2]), order, starts,
        jnp.copy(q[:, 0]), jnp.copy(q[:, 1]),
        jnp.copy(q[:, 2]),
        jnp.asarray(_CDX), jnp.asarray(_CDY), jnp.asarray(_CM2),
        jnp.asarray(_CLB2), jnp.asarray(_ZLUT), jnp.asarray(_CCNT))

    mapping = omap.reshape(_NQ, 16)[:, :_K][None]
    outputs = jnp.stack(
        [ox.reshape(_NQ, 16)[:, :_K], oy.reshape(_NQ, 16)[:, :_K],
         oz.reshape(_NQ, 16)[:, :_K]], axis=-1)[None]
    return (mapping, outputs)
